# initial kernel scaffold (unmeasured)
import jax
import jax.numpy as jnp
from jax import lax
from jax.experimental import pallas as pl
from jax.experimental.pallas import tpu as pltpu

T_PER = 1024
D = 1024
F = 4096
E_LOCAL = 8
K = 2
C = 320
FB = 512
NF = F // FB


def _peer():
    return (lax.axis_index("x"), 1 - lax.axis_index("y"), lax.axis_index("z"))


def _neighbor_barrier(peer):
    barrier_sem = pltpu.get_barrier_semaphore()
    pl.semaphore_signal(
        barrier_sem, inc=1, device_id=peer, device_id_type=pl.DeviceIdType.MESH
    )
    pl.semaphore_wait(barrier_sem, 1)


def _gather_body(x_ref, r_ref, xfull_ref, rfull_ref, send_sems, recv_sems):
    my_y = lax.axis_index("y")
    peer = _peer()
    _neighbor_barrier(peer)

    xfull_ref[pl.ds(my_y, 1)] = x_ref[...][None]
    rfull_ref[pl.ds(my_y, 1)] = r_ref[...][None]

    rdma_x = pltpu.make_async_remote_copy(
        src_ref=xfull_ref.at[my_y],
        dst_ref=xfull_ref.at[my_y],
        send_sem=send_sems.at[0],
        recv_sem=recv_sems.at[0],
        device_id=peer,
        device_id_type=pl.DeviceIdType.MESH,
    )
    rdma_r = pltpu.make_async_remote_copy(
        src_ref=rfull_ref.at[my_y],
        dst_ref=rfull_ref.at[my_y],
        send_sem=send_sems.at[1],
        recv_sem=recv_sems.at[1],
        device_id=peer,
        device_id_type=pl.DeviceIdType.MESH,
    )
    rdma_x.start()
    rdma_r.start()
    rdma_x.wait()
    rdma_r.wait()


def _gather(x, rt):
    return pl.pallas_call(
        _gather_body,
        out_shape=(
            jax.ShapeDtypeStruct((2, T_PER, D), jnp.float32),
            jax.ShapeDtypeStruct((2, E_LOCAL, D), jnp.float32),
        ),
        in_specs=[
            pl.BlockSpec(memory_space=pltpu.VMEM),
            pl.BlockSpec(memory_space=pltpu.VMEM),
        ],
        out_specs=(
            pl.BlockSpec(memory_space=pltpu.VMEM),
            pl.BlockSpec(memory_space=pltpu.VMEM),
        ),
        scratch_shapes=[
            pltpu.SemaphoreType.DMA((2,)),
            pltpu.SemaphoreType.DMA((2,)),
        ],
        compiler_params=pltpu.CompilerParams(collective_id=0),
    )(x, rt)


def _ffn_body(xg_ref, w1_ref, w2_ref, wg_ref, out_ref):
    f = pl.program_id(1)

    @pl.when(f == 0)
    def _():
        out_ref[...] = jnp.zeros_like(out_ref)

    h = jnp.dot(
        xg_ref[0],
        w1_ref[0].astype(jnp.bfloat16),
        preferred_element_type=jnp.float32,
    )
    h = jnp.maximum(h, 0.0).astype(jnp.bfloat16)
    out_ref[0] += jnp.dot(
        h, w2_ref[0].astype(jnp.bfloat16), preferred_element_type=jnp.float32
    )

    @pl.when(f == NF - 1)
    def _():
        out_ref[0] = out_ref[0] * wg_ref[0][:, None]


def _ffn(xg, W1, W2, wg):
    return pl.pallas_call(
        _ffn_body,
        grid=(E_LOCAL, NF),
        in_specs=[
            pl.BlockSpec((1, C, D), lambda e, f: (e, 0, 0)),
            pl.BlockSpec((1, D, FB), lambda e, f: (e, 0, f)),
            pl.BlockSpec((1, FB, D), lambda e, f: (e, f, 0)),
            pl.BlockSpec((1, C), lambda e, f: (e, 0)),
        ],
        out_specs=pl.BlockSpec((1, C, D), lambda e, f: (e, 0, 0)),
        out_shape=jax.ShapeDtypeStruct((E_LOCAL, C, D), jnp.float32),
    )(xg, W1, W2, wg)


def _combine_body(p_ref, out_ref, comm_ref, send_sem, recv_sem):
    my_y = lax.axis_index("y")
    peer = _peer()
    _neighbor_barrier(peer)

    rdma = pltpu.make_async_remote_copy(
        src_ref=p_ref.at[pl.ds((1 - my_y) * T_PER, T_PER)],
        dst_ref=comm_ref,
        send_sem=send_sem,
        recv_sem=recv_sem,
        device_id=peer,
        device_id_type=pl.DeviceIdType.MESH,
    )
    rdma.start()
    rdma.wait()
    out_ref[...] = p_ref[pl.ds(my_y * T_PER, T_PER), :] + comm_ref[...]


def _combine(partial):
    return pl.pallas_call(
        _combine_body,
        out_shape=jax.ShapeDtypeStruct((T_PER, D), jnp.float32),
        in_specs=[pl.BlockSpec(memory_space=pltpu.VMEM)],
        out_specs=pl.BlockSpec(memory_space=pltpu.VMEM),
        scratch_shapes=[
            pltpu.VMEM((T_PER, D), jnp.float32),
            pltpu.SemaphoreType.DMA,
            pltpu.SemaphoreType.DMA,
        ],
        compiler_params=pltpu.CompilerParams(collective_id=1),
    )(partial)


def kernel(x, router, W1, W2):
    my_y = lax.axis_index("y")

    xfull, rfull = _gather(x, router.T)
    xf = xfull.reshape(2 * T_PER, D)
    router_full = jnp.concatenate([rfull[0], rfull[1]], axis=0).T

    gates = jnp.dot(xf, router_full, precision=lax.Precision.HIGHEST)
    vals, idx = lax.top_k(gates, K)
    w = jax.nn.softmax(vals, axis=-1)

    base = my_y * E_LOCAL
    ee = idx.reshape(-1)
    ww = w.reshape(-1)
    tt = jnp.arange(2 * T_PER * K, dtype=jnp.int32) // K
    le = ee - base
    is_local = (le >= 0) & (le < E_LOCAL)
    lec = jnp.clip(le, 0, E_LOCAL - 1)
    onehot = (lec[:, None] == jnp.arange(E_LOCAL)[None, :]) & is_local[:, None]
    pos = jnp.cumsum(onehot.astype(jnp.int32), axis=0) - 1
    pos_a = jnp.take_along_axis(pos, lec[:, None], axis=1)[:, 0]
    valid = is_local & (pos_a < C)
    slot = jnp.where(valid, lec * C + pos_a, E_LOCAL * C)

    n_slots = E_LOCAL * C
    tok = jnp.zeros(n_slots + 1, jnp.int32).at[slot].set(tt)[:n_slots]
    wg = jnp.zeros(n_slots + 1, jnp.float32).at[slot].set(ww)[:n_slots]
    wg = wg.reshape(E_LOCAL, C)
    xg = xf[tok].astype(jnp.bfloat16).reshape(E_LOCAL, C, D)

    yg = _ffn(xg, W1, W2, wg)

    partial = jnp.zeros((2 * T_PER, D), jnp.float32).at[tok].add(
        yg.reshape(n_slots, D)
    )

    return _combine(partial)


# baseline (device time: 297581 ns/iter reference)
import jax
import jax.numpy as jnp
from jax import lax
from jax.experimental import pallas as pl
from jax.experimental.pallas import tpu as pltpu

T_PER = 1024
D = 1024
F = 4096
E_LOCAL = 8
K = 2
C = 320
FB = 512
NF = F // FB


def _peer():
    return (lax.axis_index("x"), 1 - lax.axis_index("y"), lax.axis_index("z"))


def _neighbor_barrier(peer):
    barrier_sem = pltpu.get_barrier_semaphore()
    pl.semaphore_signal(
        barrier_sem, inc=1, device_id=peer, device_id_type=pl.DeviceIdType.MESH
    )
    pl.semaphore_wait(barrier_sem, 1)


def _gather_body(x_ref, r_ref, xfull_ref, rfull_ref, send_sems, recv_sems):
    my_y = lax.axis_index("y")
    peer = _peer()
    _neighbor_barrier(peer)

    xfull_ref[pl.ds(my_y, 1)] = x_ref[...][None]
    rfull_ref[pl.ds(my_y, 1)] = r_ref[...][None]

    rdma_x = pltpu.make_async_remote_copy(
        src_ref=xfull_ref.at[my_y],
        dst_ref=xfull_ref.at[my_y],
        send_sem=send_sems.at[0],
        recv_sem=recv_sems.at[0],
        device_id=peer,
        device_id_type=pl.DeviceIdType.MESH,
    )
    rdma_r = pltpu.make_async_remote_copy(
        src_ref=rfull_ref.at[my_y],
        dst_ref=rfull_ref.at[my_y],
        send_sem=send_sems.at[1],
        recv_sem=recv_sems.at[1],
        device_id=peer,
        device_id_type=pl.DeviceIdType.MESH,
    )
    rdma_x.start()
    rdma_r.start()
    rdma_x.wait()
    rdma_r.wait()


def _gather(x, rt):
    return pl.pallas_call(
        _gather_body,
        out_shape=(
            jax.ShapeDtypeStruct((2, T_PER, D), jnp.float32),
            jax.ShapeDtypeStruct((2, E_LOCAL, D), jnp.float32),
        ),
        in_specs=[
            pl.BlockSpec(memory_space=pltpu.VMEM),
            pl.BlockSpec(memory_space=pltpu.VMEM),
        ],
        out_specs=(
            pl.BlockSpec(memory_space=pltpu.VMEM),
            pl.BlockSpec(memory_space=pltpu.VMEM),
        ),
        scratch_shapes=[
            pltpu.SemaphoreType.DMA((2,)),
            pltpu.SemaphoreType.DMA((2,)),
        ],
        compiler_params=pltpu.CompilerParams(collective_id=0),
    )(x, rt)


def _ffn_body(xg_ref, w1_ref, w2_ref, wg_ref, out_ref):
    f = pl.program_id(1)

    @pl.when(f == 0)
    def _():
        out_ref[...] = jnp.zeros_like(out_ref)

    h = jnp.dot(
        xg_ref[0],
        w1_ref[0].astype(jnp.bfloat16),
        preferred_element_type=jnp.float32,
    )
    h = jnp.maximum(h, 0.0).astype(jnp.bfloat16)
    out_ref[0] += jnp.dot(
        h, w2_ref[0].astype(jnp.bfloat16), preferred_element_type=jnp.float32
    )

    @pl.when(f == NF - 1)
    def _():
        out_ref[0] = out_ref[0] * wg_ref[0, 0][:, None]


def _ffn(xg, W1, W2, wg):
    return pl.pallas_call(
        _ffn_body,
        grid=(E_LOCAL, NF),
        in_specs=[
            pl.BlockSpec((1, C, D), lambda e, f: (e, 0, 0)),
            pl.BlockSpec((1, D, FB), lambda e, f: (e, 0, f)),
            pl.BlockSpec((1, FB, D), lambda e, f: (e, f, 0)),
            pl.BlockSpec((1, 1, C), lambda e, f: (e, 0, 0)),
        ],
        out_specs=pl.BlockSpec((1, C, D), lambda e, f: (e, 0, 0)),
        out_shape=jax.ShapeDtypeStruct((E_LOCAL, C, D), jnp.float32),
    )(xg, W1, W2, wg)


def _combine_body(p_ref, out_ref, comm_ref, send_sem, recv_sem):
    my_y = lax.axis_index("y")
    peer = _peer()
    _neighbor_barrier(peer)

    rdma = pltpu.make_async_remote_copy(
        src_ref=p_ref.at[pl.ds((1 - my_y) * T_PER, T_PER)],
        dst_ref=comm_ref,
        send_sem=send_sem,
        recv_sem=recv_sem,
        device_id=peer,
        device_id_type=pl.DeviceIdType.MESH,
    )
    rdma.start()
    rdma.wait()
    out_ref[...] = p_ref[pl.ds(my_y * T_PER, T_PER), :] + comm_ref[...]


def _combine(partial):
    return pl.pallas_call(
        _combine_body,
        out_shape=jax.ShapeDtypeStruct((T_PER, D), jnp.float32),
        in_specs=[pl.BlockSpec(memory_space=pltpu.VMEM)],
        out_specs=pl.BlockSpec(memory_space=pltpu.VMEM),
        scratch_shapes=[
            pltpu.VMEM((T_PER, D), jnp.float32),
            pltpu.SemaphoreType.DMA,
            pltpu.SemaphoreType.DMA,
        ],
        compiler_params=pltpu.CompilerParams(collective_id=1),
    )(partial)


def kernel(x, router, W1, W2):
    my_y = lax.axis_index("y")

    xfull, rfull = _gather(x, router.T)
    xf = xfull.reshape(2 * T_PER, D)
    router_full = jnp.concatenate([rfull[0], rfull[1]], axis=0).T

    gates = jnp.dot(xf, router_full, precision=lax.Precision.HIGHEST)
    vals, idx = lax.top_k(gates, K)
    w = jax.nn.softmax(vals, axis=-1)

    base = my_y * E_LOCAL
    ee = idx.reshape(-1)
    ww = w.reshape(-1)
    tt = jnp.arange(2 * T_PER * K, dtype=jnp.int32) // K
    le = ee - base
    is_local = (le >= 0) & (le < E_LOCAL)
    lec = jnp.clip(le, 0, E_LOCAL - 1)
    onehot = (lec[:, None] == jnp.arange(E_LOCAL)[None, :]) & is_local[:, None]
    pos = jnp.cumsum(onehot.astype(jnp.int32), axis=0) - 1
    pos_a = jnp.take_along_axis(pos, lec[:, None], axis=1)[:, 0]
    valid = is_local & (pos_a < C)
    slot = jnp.where(valid, lec * C + pos_a, E_LOCAL * C)

    n_slots = E_LOCAL * C
    tok = jnp.zeros(n_slots + 1, jnp.int32).at[slot].set(tt)[:n_slots]
    wg = jnp.zeros(n_slots + 1, jnp.float32).at[slot].set(ww)[:n_slots]
    wg = wg.reshape(E_LOCAL, 1, C)
    xg = xf[tok].astype(jnp.bfloat16).reshape(E_LOCAL, C, D)

    yg = _ffn(xg, W1, W2, wg)

    partial = jnp.zeros((2 * T_PER, D), jnp.float32).at[tok].add(
        yg.reshape(n_slots, D)
    )

    return _combine(partial)


# device time: 269163 ns/iter; 1.1056x vs baseline; 1.1056x over previous
import jax
import jax.numpy as jnp
from jax import lax
from jax.experimental import pallas as pl
from jax.experimental.pallas import tpu as pltpu

T_PER = 1024
D = 1024
F = 4096
E_LOCAL = 8
K = 2
C = 320
N_SLOTS = E_LOCAL * C
FB = 512
NF = F // FB


def _peer():
    return (lax.axis_index("x"), 1 - lax.axis_index("y"), lax.axis_index("z"))


def _neighbor_barrier(peer):
    barrier_sem = pltpu.get_barrier_semaphore()
    pl.semaphore_signal(
        barrier_sem, inc=1, device_id=peer, device_id_type=pl.DeviceIdType.MESH
    )
    pl.semaphore_wait(barrier_sem, 1)


def _gather_body(x_ref, r_ref, xfull_ref, rfull_ref, send_sems, recv_sems):
    my_y = lax.axis_index("y")
    peer = _peer()
    _neighbor_barrier(peer)

    xfull_ref[pl.ds(my_y, 1)] = x_ref[...][None]
    rfull_ref[pl.ds(my_y, 1)] = r_ref[...][None]

    rdma_x = pltpu.make_async_remote_copy(
        src_ref=xfull_ref.at[my_y],
        dst_ref=xfull_ref.at[my_y],
        send_sem=send_sems.at[0],
        recv_sem=recv_sems.at[0],
        device_id=peer,
        device_id_type=pl.DeviceIdType.MESH,
    )
    rdma_r = pltpu.make_async_remote_copy(
        src_ref=rfull_ref.at[my_y],
        dst_ref=rfull_ref.at[my_y],
        send_sem=send_sems.at[1],
        recv_sem=recv_sems.at[1],
        device_id=peer,
        device_id_type=pl.DeviceIdType.MESH,
    )
    rdma_x.start()
    rdma_r.start()
    rdma_x.wait()
    rdma_r.wait()


def _gather(x, rt):
    return pl.pallas_call(
        _gather_body,
        out_shape=(
            jax.ShapeDtypeStruct((2, T_PER, D), jnp.float32),
            jax.ShapeDtypeStruct((2, E_LOCAL, D), jnp.float32),
        ),
        in_specs=[
            pl.BlockSpec(memory_space=pltpu.VMEM),
            pl.BlockSpec(memory_space=pltpu.VMEM),
        ],
        out_specs=(
            pl.BlockSpec(memory_space=pltpu.VMEM),
            pl.BlockSpec(memory_space=pltpu.VMEM),
        ),
        scratch_shapes=[
            pltpu.SemaphoreType.DMA((2,)),
            pltpu.SemaphoreType.DMA((2,)),
        ],
        compiler_params=pltpu.CompilerParams(collective_id=0),
    )(x, rt)


def _ffn_body(xf_ref, tok_ref, w1_ref, w2_ref, wg_ref, out_ref, xg_ref, acc_ref):
    f = pl.program_id(1)

    @pl.when(f == 0)
    def _():
        tokb = tok_ref[0, 0]
        oh = (
            tokb[:, None]
            == lax.broadcasted_iota(jnp.int32, (C, 2 * T_PER), 1)
        ).astype(jnp.bfloat16)
        xg_ref[...] = jnp.dot(
            oh, xf_ref[...], preferred_element_type=jnp.float32
        ).astype(jnp.bfloat16)
        acc_ref[...] = jnp.zeros_like(acc_ref)

    h = jnp.dot(
        xg_ref[...],
        w1_ref[0].astype(jnp.bfloat16),
        preferred_element_type=jnp.float32,
    )
    h = jnp.maximum(h, 0.0).astype(jnp.bfloat16)
    acc_ref[...] += jnp.dot(
        h, w2_ref[0].astype(jnp.bfloat16), preferred_element_type=jnp.float32
    )

    @pl.when(f == NF - 1)
    def _():
        out_ref[0] = (acc_ref[...] * wg_ref[0, 0][:, None]).astype(jnp.bfloat16)


def _ffn(xf_bf, tok, W1, W2, wg):
    return pl.pallas_call(
        _ffn_body,
        grid=(E_LOCAL, NF),
        in_specs=[
            pl.BlockSpec((2 * T_PER, D), lambda e, f: (0, 0)),
            pl.BlockSpec((1, 1, C), lambda e, f: (e, 0, 0)),
            pl.BlockSpec((1, D, FB), lambda e, f: (e, 0, f)),
            pl.BlockSpec((1, FB, D), lambda e, f: (e, f, 0)),
            pl.BlockSpec((1, 1, C), lambda e, f: (e, 0, 0)),
        ],
        out_specs=pl.BlockSpec((1, C, D), lambda e, f: (e, 0, 0)),
        out_shape=jax.ShapeDtypeStruct((E_LOCAL, C, D), jnp.bfloat16),
        scratch_shapes=[
            pltpu.VMEM((C, D), jnp.bfloat16),
            pltpu.VMEM((C, D), jnp.float32),
        ],
    )(xf_bf, tok, W1, W2, wg)


def _combine_body(
    y_ref, s2_ref, out_ref, partial_ref, send_buf, comm_ref, send_sem, recv_sem
):
    my_y = lax.axis_index("y")
    peer = _peer()

    n_chunk = 4
    rows = (2 * T_PER) // n_chunk
    for i in range(n_chunk):
        st0 = s2_ref[0, pl.ds(i * rows, rows)]
        st1 = s2_ref[1, pl.ds(i * rows, rows)]
        iota = lax.broadcasted_iota(jnp.int32, (rows, N_SLOTS), 1)
        oh2 = (st0[:, None] == iota).astype(jnp.bfloat16) + (
            st1[:, None] == iota
        ).astype(jnp.bfloat16)
        partial_ref[pl.ds(i * rows, rows), :] = jnp.dot(
            oh2, y_ref[...], preferred_element_type=jnp.float32
        )

    send_buf[...] = partial_ref[pl.ds((1 - my_y) * T_PER, T_PER), :].astype(
        jnp.bfloat16
    )
    _neighbor_barrier(peer)
    rdma = pltpu.make_async_remote_copy(
        src_ref=send_buf,
        dst_ref=comm_ref,
        send_sem=send_sem,
        recv_sem=recv_sem,
        device_id=peer,
        device_id_type=pl.DeviceIdType.MESH,
    )
    rdma.start()
    rdma.wait()
    out_ref[...] = partial_ref[pl.ds(my_y * T_PER, T_PER), :] + comm_ref[
        ...
    ].astype(jnp.float32)


def _combine(yflat, s2):
    return pl.pallas_call(
        _combine_body,
        out_shape=jax.ShapeDtypeStruct((T_PER, D), jnp.float32),
        in_specs=[
            pl.BlockSpec(memory_space=pltpu.VMEM),
            pl.BlockSpec(memory_space=pltpu.VMEM),
        ],
        out_specs=pl.BlockSpec(memory_space=pltpu.VMEM),
        scratch_shapes=[
            pltpu.VMEM((2 * T_PER, D), jnp.float32),
            pltpu.VMEM((T_PER, D), jnp.bfloat16),
            pltpu.VMEM((T_PER, D), jnp.bfloat16),
            pltpu.SemaphoreType.DMA,
            pltpu.SemaphoreType.DMA,
        ],
        compiler_params=pltpu.CompilerParams(collective_id=1),
    )(yflat, s2)


def kernel(x, router, W1, W2):
    my_y = lax.axis_index("y")

    xfull, rfull = _gather(x, router.T)
    xf = xfull.reshape(2 * T_PER, D)
    router_full = jnp.concatenate([rfull[0], rfull[1]], axis=0).T

    gates = jnp.dot(xf, router_full, precision=lax.Precision.HIGHEST)
    vals, idx = lax.top_k(gates, K)
    w = jax.nn.softmax(vals, axis=-1)

    base = my_y * E_LOCAL
    ee = idx.reshape(-1)
    ww = w.reshape(-1)
    tt = jnp.arange(2 * T_PER * K, dtype=jnp.int32) // K
    le = ee - base
    is_local = (le >= 0) & (le < E_LOCAL)
    lec = jnp.clip(le, 0, E_LOCAL - 1)
    onehot = (lec[:, None] == jnp.arange(E_LOCAL)[None, :]) & is_local[:, None]
    pos = jnp.cumsum(onehot.astype(jnp.int32), axis=0) - 1
    pos_a = jnp.take_along_axis(pos, lec[:, None], axis=1)[:, 0]
    valid = is_local & (pos_a < C)
    slot = jnp.where(valid, lec * C + pos_a, N_SLOTS)

    tok = jnp.zeros(N_SLOTS + 1, jnp.int32).at[slot].set(tt)[:N_SLOTS]
    tok = tok.reshape(E_LOCAL, 1, C)
    wg = jnp.zeros(N_SLOTS + 1, jnp.float32).at[slot].set(ww)[:N_SLOTS]
    wg = wg.reshape(E_LOCAL, 1, C)
    s2 = slot.reshape(2 * T_PER, K).T

    yg = _ffn(xf.astype(jnp.bfloat16), tok, W1, W2, wg)

    return _combine(yg.reshape(N_SLOTS, D), s2)


# device time: 242005 ns/iter; 1.2296x vs baseline; 1.1122x over previous
import jax
import jax.numpy as jnp
from jax import lax
from jax.experimental import pallas as pl
from jax.experimental.pallas import tpu as pltpu

T_PER = 1024
D = 1024
F = 4096
E_LOCAL = 8
K = 2
C = 320
N_SLOTS = E_LOCAL * C
FB = 512
NF = F // FB


def _peer():
    return (lax.axis_index("x"), 1 - lax.axis_index("y"), lax.axis_index("z"))


def _neighbor_barrier(peer):
    barrier_sem = pltpu.get_barrier_semaphore()
    pl.semaphore_signal(
        barrier_sem, inc=1, device_id=peer, device_id_type=pl.DeviceIdType.MESH
    )
    pl.semaphore_wait(barrier_sem, 1)


def _gather_body(x_ref, r_ref, xfull_ref, rfull_ref, send_sems, recv_sems):
    my_y = lax.axis_index("y")
    peer = _peer()
    _neighbor_barrier(peer)

    xfull_ref[pl.ds(my_y, 1)] = x_ref[...][None]
    rfull_ref[pl.ds(my_y, 1)] = r_ref[...][None]

    rdma_x = pltpu.make_async_remote_copy(
        src_ref=xfull_ref.at[my_y],
        dst_ref=xfull_ref.at[my_y],
        send_sem=send_sems.at[0],
        recv_sem=recv_sems.at[0],
        device_id=peer,
        device_id_type=pl.DeviceIdType.MESH,
    )
    rdma_r = pltpu.make_async_remote_copy(
        src_ref=rfull_ref.at[my_y],
        dst_ref=rfull_ref.at[my_y],
        send_sem=send_sems.at[1],
        recv_sem=recv_sems.at[1],
        device_id=peer,
        device_id_type=pl.DeviceIdType.MESH,
    )
    rdma_x.start()
    rdma_r.start()
    rdma_x.wait()
    rdma_r.wait()


def _gather(x, rt):
    return pl.pallas_call(
        _gather_body,
        out_shape=(
            jax.ShapeDtypeStruct((2, T_PER, D), jnp.float32),
            jax.ShapeDtypeStruct((2, E_LOCAL, D), jnp.float32),
        ),
        in_specs=[
            pl.BlockSpec(memory_space=pltpu.VMEM),
            pl.BlockSpec(memory_space=pltpu.VMEM),
        ],
        out_specs=(
            pl.BlockSpec(memory_space=pltpu.VMEM),
            pl.BlockSpec(memory_space=pltpu.VMEM),
        ),
        scratch_shapes=[
            pltpu.SemaphoreType.DMA((2,)),
            pltpu.SemaphoreType.DMA((2,)),
        ],
        compiler_params=pltpu.CompilerParams(collective_id=0),
    )(x, rt)


A = 2 * T_PER * K


def _ffn_body(xf_ref, slot_ref, ww_ref, w1_ref, w2_ref, out_ref, xg_ref,
              acc_ref, wg_ref):
    e = pl.program_id(0)
    f = pl.program_id(1)

    @pl.when(f == 0)
    def _():
        sl = slot_ref[0]
        my_slots = lax.broadcasted_iota(jnp.int32, (C, A), 0) + e * C
        ohs = my_slots == sl[None, :]
        ta = lax.broadcasted_iota(jnp.int32, (C, A), 1) // K
        tok = jnp.sum(jnp.where(ohs, ta, 0), axis=1)
        wg_ref[0, :] = jnp.sum(
            jnp.where(ohs, ww_ref[0][None, :], 0.0), axis=1
        )
        oh = (
            tok[:, None]
            == lax.broadcasted_iota(jnp.int32, (C, 2 * T_PER), 1)
        ).astype(jnp.bfloat16)
        xg_ref[...] = jnp.dot(
            oh, xf_ref[...], preferred_element_type=jnp.float32
        ).astype(jnp.bfloat16)
        acc_ref[...] = jnp.zeros_like(acc_ref)

    h = jnp.dot(
        xg_ref[...],
        w1_ref[0].astype(jnp.bfloat16),
        preferred_element_type=jnp.float32,
    )
    h = jnp.maximum(h, 0.0).astype(jnp.bfloat16)
    acc_ref[...] += jnp.dot(
        h, w2_ref[0].astype(jnp.bfloat16), preferred_element_type=jnp.float32
    )

    @pl.when(f == NF - 1)
    def _():
        out_ref[0] = (acc_ref[...] * wg_ref[0, :][:, None]).astype(jnp.bfloat16)


def _ffn(xf_bf, slot, ww, W1, W2):
    return pl.pallas_call(
        _ffn_body,
        grid=(E_LOCAL, NF),
        in_specs=[
            pl.BlockSpec((2 * T_PER, D), lambda e, f: (0, 0)),
            pl.BlockSpec((1, A), lambda e, f: (0, 0)),
            pl.BlockSpec((1, A), lambda e, f: (0, 0)),
            pl.BlockSpec((1, D, FB), lambda e, f: (e, 0, f)),
            pl.BlockSpec((1, FB, D), lambda e, f: (e, f, 0)),
        ],
        out_specs=pl.BlockSpec((1, C, D), lambda e, f: (e, 0, 0)),
        out_shape=jax.ShapeDtypeStruct((E_LOCAL, C, D), jnp.bfloat16),
        scratch_shapes=[
            pltpu.VMEM((C, D), jnp.bfloat16),
            pltpu.VMEM((C, D), jnp.float32),
            pltpu.VMEM((1, C), jnp.float32),
        ],
    )(xf_bf, slot, ww, W1, W2)


def _combine_body(
    y_ref, s2_ref, out_ref, partial_ref, send_buf, comm_ref, send_sem, recv_sem
):
    my_y = lax.axis_index("y")
    peer = _peer()

    n_chunk = 4
    rows = (2 * T_PER) // n_chunk
    for i in range(n_chunk):
        st0 = s2_ref[0, pl.ds(i * rows, rows)]
        st1 = s2_ref[1, pl.ds(i * rows, rows)]
        iota = lax.broadcasted_iota(jnp.int32, (rows, N_SLOTS), 1)
        oh2 = (st0[:, None] == iota).astype(jnp.bfloat16) + (
            st1[:, None] == iota
        ).astype(jnp.bfloat16)
        partial_ref[pl.ds(i * rows, rows), :] = jnp.dot(
            oh2, y_ref[...], preferred_element_type=jnp.float32
        )

    send_buf[...] = partial_ref[pl.ds((1 - my_y) * T_PER, T_PER), :].astype(
        jnp.bfloat16
    )
    _neighbor_barrier(peer)
    rdma = pltpu.make_async_remote_copy(
        src_ref=send_buf,
        dst_ref=comm_ref,
        send_sem=send_sem,
        recv_sem=recv_sem,
        device_id=peer,
        device_id_type=pl.DeviceIdType.MESH,
    )
    rdma.start()
    rdma.wait()
    out_ref[...] = partial_ref[pl.ds(my_y * T_PER, T_PER), :] + comm_ref[
        ...
    ].astype(jnp.float32)


def _combine(yflat, s2):
    return pl.pallas_call(
        _combine_body,
        out_shape=jax.ShapeDtypeStruct((T_PER, D), jnp.float32),
        in_specs=[
            pl.BlockSpec(memory_space=pltpu.VMEM),
            pl.BlockSpec(memory_space=pltpu.VMEM),
        ],
        out_specs=pl.BlockSpec(memory_space=pltpu.VMEM),
        scratch_shapes=[
            pltpu.VMEM((2 * T_PER, D), jnp.float32),
            pltpu.VMEM((T_PER, D), jnp.bfloat16),
            pltpu.VMEM((T_PER, D), jnp.bfloat16),
            pltpu.SemaphoreType.DMA,
            pltpu.SemaphoreType.DMA,
        ],
        compiler_params=pltpu.CompilerParams(collective_id=1),
    )(yflat, s2)


def kernel(x, router, W1, W2):
    my_y = lax.axis_index("y")

    xfull, rfull = _gather(x, router.T)
    xf = xfull.reshape(2 * T_PER, D)
    router_full = jnp.concatenate([rfull[0], rfull[1]], axis=0).T

    gates = jnp.dot(xf, router_full, precision=lax.Precision.HIGHEST)
    vals, idx = lax.top_k(gates, K)
    w = jax.nn.softmax(vals, axis=-1)

    base = my_y * E_LOCAL
    ee = idx.reshape(-1)
    ww = w.reshape(-1)
    tt = jnp.arange(2 * T_PER * K, dtype=jnp.int32) // K
    le = ee - base
    is_local = (le >= 0) & (le < E_LOCAL)
    lec = jnp.clip(le, 0, E_LOCAL - 1)
    onehot = (lec[:, None] == jnp.arange(E_LOCAL)[None, :]) & is_local[:, None]
    pos = jnp.cumsum(onehot.astype(jnp.int32), axis=0) - 1
    pos_a = jnp.sum(jnp.where(onehot, pos, 0), axis=1)
    valid = is_local & (pos_a < C)
    slot = jnp.where(valid, lec * C + pos_a, N_SLOTS)
    s2 = slot.reshape(2 * T_PER, K).T

    yg = _ffn(xf.astype(jnp.bfloat16), slot[None, :], ww[None, :], W1, W2)

    return _combine(yg.reshape(N_SLOTS, D), s2)


# device time: 217591 ns/iter; 1.3676x vs baseline; 1.1122x over previous
import jax
import jax.numpy as jnp
from jax import lax
from jax.experimental import pallas as pl
from jax.experimental.pallas import tpu as pltpu

T_PER = 1024
D = 1024
F = 4096
E_LOCAL = 8
K = 2
C = 320
N_SLOTS = E_LOCAL * C
FB = 512
NF = F // FB


def _peer():
    return (lax.axis_index("x"), 1 - lax.axis_index("y"), lax.axis_index("z"))


def _neighbor_barrier(peer):
    barrier_sem = pltpu.get_barrier_semaphore()
    pl.semaphore_signal(
        barrier_sem, inc=1, device_id=peer, device_id_type=pl.DeviceIdType.MESH
    )
    pl.semaphore_wait(barrier_sem, 1)


def _rexch_body(r_ref, rfull_ref, send_sem, recv_sem):
    my_y = lax.axis_index("y")
    peer = _peer()
    _neighbor_barrier(peer)

    rfull_ref[pl.ds(my_y, 1)] = r_ref[...][None]
    rdma = pltpu.make_async_remote_copy(
        src_ref=rfull_ref.at[my_y],
        dst_ref=rfull_ref.at[my_y],
        send_sem=send_sem,
        recv_sem=recv_sem,
        device_id=peer,
        device_id_type=pl.DeviceIdType.MESH,
    )
    rdma.start()
    rdma.wait()


def _rexch(rt):
    return pl.pallas_call(
        _rexch_body,
        out_shape=jax.ShapeDtypeStruct((2, E_LOCAL, D), jnp.float32),
        in_specs=[pl.BlockSpec(memory_space=pltpu.VMEM)],
        out_specs=pl.BlockSpec(memory_space=pltpu.VMEM),
        scratch_shapes=[pltpu.SemaphoreType.DMA, pltpu.SemaphoreType.DMA],
        compiler_params=pltpu.CompilerParams(collective_id=0),
    )(rt)


def _xexch_body(
    xb_ref, idx_ref, w_ref, xfull_ref, idxfull_ref, wfull_ref,
    send_sems, recv_sems
):
    my_y = lax.axis_index("y")
    peer = _peer()
    _neighbor_barrier(peer)

    xfull_ref[pl.ds(my_y, 1)] = xb_ref[...][None]
    idxfull_ref[pl.ds(my_y, 1)] = idx_ref[...][None]
    wfull_ref[pl.ds(my_y, 1)] = w_ref[...][None]

    rdmas = [
        pltpu.make_async_remote_copy(
            src_ref=ref.at[my_y],
            dst_ref=ref.at[my_y],
            send_sem=send_sems.at[i],
            recv_sem=recv_sems.at[i],
            device_id=peer,
            device_id_type=pl.DeviceIdType.MESH,
        )
        for i, ref in enumerate([xfull_ref, idxfull_ref, wfull_ref])
    ]
    for r in rdmas:
        r.start()
    for r in rdmas:
        r.wait()


def _xexch(xb, idx, w):
    return pl.pallas_call(
        _xexch_body,
        out_shape=(
            jax.ShapeDtypeStruct((2, T_PER, D), jnp.bfloat16),
            jax.ShapeDtypeStruct((2, K, T_PER), jnp.int32),
            jax.ShapeDtypeStruct((2, K, T_PER), jnp.float32),
        ),
        in_specs=[pl.BlockSpec(memory_space=pltpu.VMEM)] * 3,
        out_specs=(pl.BlockSpec(memory_space=pltpu.VMEM),) * 3,
        scratch_shapes=[
            pltpu.SemaphoreType.DMA((3,)),
            pltpu.SemaphoreType.DMA((3,)),
        ],
        compiler_params=pltpu.CompilerParams(collective_id=2),
    )(xb, idx, w)


A = 2 * T_PER * K


def _ffn_body(xf_ref, slot_ref, ww_ref, w1_ref, w2_ref, out_ref, xg_ref,
              acc_ref, wg_ref):
    e = pl.program_id(0)
    f = pl.program_id(1)

    @pl.when(f == 0)
    def _():
        sl = slot_ref[0]
        my_slots = lax.broadcasted_iota(jnp.int32, (C, A), 0) + e * C
        ohs = my_slots == sl[None, :]
        ta = lax.broadcasted_iota(jnp.int32, (C, A), 1) // K
        tok = jnp.sum(jnp.where(ohs, ta, 0), axis=1)
        wg_ref[0, :] = jnp.sum(
            jnp.where(ohs, ww_ref[0][None, :], 0.0), axis=1
        )
        oh = (
            tok[:, None]
            == lax.broadcasted_iota(jnp.int32, (C, 2 * T_PER), 1)
        ).astype(jnp.bfloat16)
        xg_ref[...] = jnp.dot(
            oh, xf_ref[...], preferred_element_type=jnp.float32
        ).astype(jnp.bfloat16)
        acc_ref[...] = jnp.zeros_like(acc_ref)

    h = jnp.dot(
        xg_ref[...],
        w1_ref[0].astype(jnp.bfloat16),
        preferred_element_type=jnp.float32,
    )
    h = jnp.maximum(h, 0.0).astype(jnp.bfloat16)
    acc_ref[...] += jnp.dot(
        h, w2_ref[0].astype(jnp.bfloat16), preferred_element_type=jnp.float32
    )

    @pl.when(f == NF - 1)
    def _():
        out_ref[0] = (acc_ref[...] * wg_ref[0, :][:, None]).astype(jnp.bfloat16)


def _ffn(xf_bf, slot, ww, W1, W2):
    return pl.pallas_call(
        _ffn_body,
        grid=(E_LOCAL, NF),
        in_specs=[
            pl.BlockSpec((2 * T_PER, D), lambda e, f: (0, 0)),
            pl.BlockSpec((1, A), lambda e, f: (0, 0)),
            pl.BlockSpec((1, A), lambda e, f: (0, 0)),
            pl.BlockSpec((1, D, FB), lambda e, f: (e, 0, f)),
            pl.BlockSpec((1, FB, D), lambda e, f: (e, f, 0)),
        ],
        out_specs=pl.BlockSpec((1, C, D), lambda e, f: (e, 0, 0)),
        out_shape=jax.ShapeDtypeStruct((E_LOCAL, C, D), jnp.bfloat16),
        scratch_shapes=[
            pltpu.VMEM((C, D), jnp.bfloat16),
            pltpu.VMEM((C, D), jnp.float32),
            pltpu.VMEM((1, C), jnp.float32),
        ],
    )(xf_bf, slot, ww, W1, W2)


def _combine_body(
    y_ref, s2_ref, out_ref, partial_ref, send_buf, comm_ref, send_sem, recv_sem
):
    my_y = lax.axis_index("y")
    peer = _peer()

    n_chunk = 4
    rows = (2 * T_PER) // n_chunk
    for i in range(n_chunk):
        st0 = s2_ref[0, pl.ds(i * rows, rows)]
        st1 = s2_ref[1, pl.ds(i * rows, rows)]
        iota = lax.broadcasted_iota(jnp.int32, (rows, N_SLOTS), 1)
        oh2 = (st0[:, None] == iota).astype(jnp.bfloat16) + (
            st1[:, None] == iota
        ).astype(jnp.bfloat16)
        partial_ref[pl.ds(i * rows, rows), :] = jnp.dot(
            oh2, y_ref[...], preferred_element_type=jnp.float32
        )

    send_buf[...] = partial_ref[pl.ds((1 - my_y) * T_PER, T_PER), :].astype(
        jnp.bfloat16
    )
    _neighbor_barrier(peer)
    rdma = pltpu.make_async_remote_copy(
        src_ref=send_buf,
        dst_ref=comm_ref,
        send_sem=send_sem,
        recv_sem=recv_sem,
        device_id=peer,
        device_id_type=pl.DeviceIdType.MESH,
    )
    rdma.start()
    rdma.wait()
    out_ref[...] = partial_ref[pl.ds(my_y * T_PER, T_PER), :] + comm_ref[
        ...
    ].astype(jnp.float32)


def _combine(yflat, s2):
    return pl.pallas_call(
        _combine_body,
        out_shape=jax.ShapeDtypeStruct((T_PER, D), jnp.float32),
        in_specs=[
            pl.BlockSpec(memory_space=pltpu.VMEM),
            pl.BlockSpec(memory_space=pltpu.VMEM),
        ],
        out_specs=pl.BlockSpec(memory_space=pltpu.VMEM),
        scratch_shapes=[
            pltpu.VMEM((2 * T_PER, D), jnp.float32),
            pltpu.VMEM((T_PER, D), jnp.bfloat16),
            pltpu.VMEM((T_PER, D), jnp.bfloat16),
            pltpu.SemaphoreType.DMA,
            pltpu.SemaphoreType.DMA,
        ],
        compiler_params=pltpu.CompilerParams(collective_id=1),
    )(yflat, s2)


def kernel(x, router, W1, W2):
    my_y = lax.axis_index("y")

    rfull = _rexch(router.T)
    router_full = jnp.concatenate([rfull[0], rfull[1]], axis=0).T
    gates = jnp.dot(x, router_full, precision=lax.Precision.HIGHEST)
    vals, idx = lax.top_k(gates, K)
    w = jax.nn.softmax(vals, axis=-1)

    xfull, idxfull, wfull = _xexch(
        x.astype(jnp.bfloat16), idx.T, w.T.astype(jnp.float32)
    )
    xf = xfull.reshape(2 * T_PER, D)
    idx_g = jnp.concatenate([idxfull[0].T, idxfull[1].T], axis=0)
    w_g = jnp.concatenate([wfull[0].T, wfull[1].T], axis=0)

    base = my_y * E_LOCAL
    ee = idx_g.reshape(-1)
    ww = w_g.reshape(-1)
    le = ee - base
    is_local = (le >= 0) & (le < E_LOCAL)
    lec = jnp.clip(le, 0, E_LOCAL - 1)
    onehot = (lec[:, None] == jnp.arange(E_LOCAL)[None, :]) & is_local[:, None]
    pos = jnp.cumsum(onehot.astype(jnp.int32), axis=0) - 1
    pos_a = jnp.sum(jnp.where(onehot, pos, 0), axis=1)
    valid = is_local & (pos_a < C)
    slot = jnp.where(valid, lec * C + pos_a, N_SLOTS)
    s2 = slot.reshape(2 * T_PER, K).T

    yg = _ffn(xf, slot[None, :], ww[None, :], W1, W2)

    return _combine(yg.reshape(N_SLOTS, D), s2)


# device time: 181158 ns/iter; 1.6427x vs baseline; 1.2011x over previous
import jax
import jax.numpy as jnp
from jax import lax
from jax.experimental import pallas as pl
from jax.experimental.pallas import tpu as pltpu

T_PER = 1024
D = 1024
F = 4096
E_LOCAL = 8
K = 2
C = 320
N_SLOTS = E_LOCAL * C
FB = 512
NF = F // FB


def _peer():
    return (lax.axis_index("x"), 1 - lax.axis_index("y"), lax.axis_index("z"))


def _neighbor_barrier(peer):
    barrier_sem = pltpu.get_barrier_semaphore()
    pl.semaphore_signal(
        barrier_sem, inc=1, device_id=peer, device_id_type=pl.DeviceIdType.MESH
    )
    pl.semaphore_wait(barrier_sem, 1)


def _rexch_body(r_ref, rfull_ref, send_sem, recv_sem):
    my_y = lax.axis_index("y")
    peer = _peer()
    _neighbor_barrier(peer)

    rfull_ref[pl.ds(my_y, 1)] = r_ref[...][None]
    rdma = pltpu.make_async_remote_copy(
        src_ref=rfull_ref.at[my_y],
        dst_ref=rfull_ref.at[my_y],
        send_sem=send_sem,
        recv_sem=recv_sem,
        device_id=peer,
        device_id_type=pl.DeviceIdType.MESH,
    )
    rdma.start()
    rdma.wait()


def _rexch(rt):
    return pl.pallas_call(
        _rexch_body,
        out_shape=jax.ShapeDtypeStruct((2, E_LOCAL, D), jnp.float32),
        in_specs=[pl.BlockSpec(memory_space=pltpu.VMEM)],
        out_specs=pl.BlockSpec(memory_space=pltpu.VMEM),
        scratch_shapes=[pltpu.SemaphoreType.DMA, pltpu.SemaphoreType.DMA],
        compiler_params=pltpu.CompilerParams(collective_id=0),
    )(rt)


def _xexch_body(
    xb_ref, idx_ref, w_ref, xfull_ref, idxfull_ref, wfull_ref,
    send_sems, recv_sems
):
    my_y = lax.axis_index("y")
    peer = _peer()
    _neighbor_barrier(peer)

    xfull_ref[pl.ds(my_y, 1)] = xb_ref[...][None]
    idxfull_ref[pl.ds(my_y, 1)] = idx_ref[...][None]
    wfull_ref[pl.ds(my_y, 1)] = w_ref[...][None]

    rdmas = [
        pltpu.make_async_remote_copy(
            src_ref=ref.at[my_y],
            dst_ref=ref.at[my_y],
            send_sem=send_sems.at[i],
            recv_sem=recv_sems.at[i],
            device_id=peer,
            device_id_type=pl.DeviceIdType.MESH,
        )
        for i, ref in enumerate([xfull_ref, idxfull_ref, wfull_ref])
    ]
    for r in rdmas:
        r.start()
    for r in rdmas:
        r.wait()


def _xexch(xb, idx, w):
    return pl.pallas_call(
        _xexch_body,
        out_shape=(
            jax.ShapeDtypeStruct((2, T_PER, D), jnp.bfloat16),
            jax.ShapeDtypeStruct((2, K, T_PER), jnp.int32),
            jax.ShapeDtypeStruct((2, K, T_PER), jnp.float32),
        ),
        in_specs=[pl.BlockSpec(memory_space=pltpu.VMEM)] * 3,
        out_specs=(pl.BlockSpec(memory_space=pltpu.VMEM),) * 3,
        scratch_shapes=[
            pltpu.SemaphoreType.DMA((3,)),
            pltpu.SemaphoreType.DMA((3,)),
        ],
        compiler_params=pltpu.CompilerParams(collective_id=2),
    )(xb, idx, w)


A = 2 * T_PER * K

N_FP = 4
NF_PER = NF // N_FP


def _ffn_body(fp_ref, xf_ref, slot_ref, ww_ref, w1_ref, w2_ref, out_ref,
              xg_ref, acc_ref, wg_ref):
    e = pl.program_id(0)
    f = pl.program_id(1)

    @pl.when(f == 0)
    def _():
        sl = slot_ref[0]
        my_slots = lax.broadcasted_iota(jnp.int32, (C, A), 0) + e * C
        ohs = my_slots == sl[None, :]
        ta = lax.broadcasted_iota(jnp.int32, (C, A), 1) // K
        tok = jnp.sum(jnp.where(ohs, ta, 0), axis=1)
        wg_ref[0, :] = jnp.sum(
            jnp.where(ohs, ww_ref[0][None, :], 0.0), axis=1
        )
        oh = (
            tok[:, None]
            == lax.broadcasted_iota(jnp.int32, (C, 2 * T_PER), 1)
        ).astype(jnp.bfloat16)
        xg_ref[...] = jnp.dot(
            oh, xf_ref[...], preferred_element_type=jnp.float32
        ).astype(jnp.bfloat16)
        acc_ref[...] = jnp.zeros_like(acc_ref)

    h = jnp.dot(
        xg_ref[...],
        w1_ref[0].astype(jnp.bfloat16),
        preferred_element_type=jnp.float32,
    )
    h = jnp.maximum(h, 0.0).astype(jnp.bfloat16)
    acc_ref[...] += jnp.dot(
        h, w2_ref[0].astype(jnp.bfloat16), preferred_element_type=jnp.float32
    )

    @pl.when(f == NF_PER - 1)
    def _():
        out_ref[0] = (acc_ref[...] * wg_ref[0, :][:, None]).astype(jnp.bfloat16)


def _ffn(fp, xf_bf, slot, ww, W1, W2):
    grid_spec = pltpu.PrefetchScalarGridSpec(
        num_scalar_prefetch=1,
        grid=(E_LOCAL, NF_PER),
        in_specs=[
            pl.BlockSpec((2 * T_PER, D), lambda e, f, fp: (0, 0)),
            pl.BlockSpec((1, A), lambda e, f, fp: (0, 0)),
            pl.BlockSpec((1, A), lambda e, f, fp: (0, 0)),
            pl.BlockSpec((1, D, FB), lambda e, f, fp: (e, 0, fp[0] * NF_PER + f)),
            pl.BlockSpec((1, FB, D), lambda e, f, fp: (e, fp[0] * NF_PER + f, 0)),
        ],
        out_specs=pl.BlockSpec((1, C, D), lambda e, f, fp: (e, 0, 0)),
        scratch_shapes=[
            pltpu.VMEM((C, D), jnp.bfloat16),
            pltpu.VMEM((C, D), jnp.float32),
            pltpu.VMEM((1, C), jnp.float32),
        ],
    )
    return pl.pallas_call(
        _ffn_body,
        grid_spec=grid_spec,
        out_shape=jax.ShapeDtypeStruct((E_LOCAL, C, D), jnp.bfloat16),
    )(fp, xf_bf, slot, ww, W1, W2)


def _undispatch_rows(s2_ref, y_ref, base, dst_ref, to_bf16):
    n_chunk = 2
    rows = T_PER // n_chunk
    for i in range(n_chunk):
        st0 = s2_ref[0, pl.ds(base + i * rows, rows)]
        st1 = s2_ref[1, pl.ds(base + i * rows, rows)]
        iota = lax.broadcasted_iota(jnp.int32, (rows, N_SLOTS), 1)
        oh2 = (st0[:, None] == iota).astype(jnp.bfloat16) + (
            st1[:, None] == iota
        ).astype(jnp.bfloat16)
        part = jnp.dot(oh2, y_ref[...], preferred_element_type=jnp.float32)
        dst_ref[pl.ds(i * rows, rows), :] = (
            part.astype(jnp.bfloat16) if to_bf16 else part
        )


def _combine_body(
    y_ref, s2_ref, out_ref, acc_ref, send_ref, rbuf_y, rbuf_x, rbuf_z,
    send_sems, recv_sems
):
    my_x = lax.axis_index("x")
    my_y = lax.axis_index("y")
    my_z = lax.axis_index("z")
    n_y = (my_x, 1 - my_y, my_z)
    n_x = (1 - my_x, my_y, my_z)
    n_z = (my_x, my_y, 1 - my_z)

    barrier_sem = pltpu.get_barrier_semaphore()
    for nbr in (n_y, n_x, n_z):
        pl.semaphore_signal(
            barrier_sem, inc=1, device_id=nbr,
            device_id_type=pl.DeviceIdType.MESH,
        )
    pl.semaphore_wait(barrier_sem, 3)

    def exchange(i, dst, nbr):
        return pltpu.make_async_remote_copy(
            src_ref=send_ref,
            dst_ref=dst,
            send_sem=send_sems.at[i],
            recv_sem=recv_sems.at[i],
            device_id=nbr,
            device_id_type=pl.DeviceIdType.MESH,
        )

    _undispatch_rows(s2_ref, y_ref, (1 - my_y) * T_PER, send_ref, True)
    rd_y = exchange(0, rbuf_y, n_y)
    rd_y.start()
    _undispatch_rows(s2_ref, y_ref, my_y * T_PER, acc_ref, False)
    rd_y.wait()
    acc_ref[...] += rbuf_y[...].astype(jnp.float32)

    send_ref[...] = acc_ref[...].astype(jnp.bfloat16)
    rd_x = exchange(1, rbuf_x, n_x)
    rd_x.start()
    rd_x.wait()
    acc_ref[...] += rbuf_x[...].astype(jnp.float32)

    send_ref[...] = acc_ref[...].astype(jnp.bfloat16)
    rd_z = exchange(2, rbuf_z, n_z)
    rd_z.start()
    rd_z.wait()
    out_ref[...] = acc_ref[...] + rbuf_z[...].astype(jnp.float32)


def _combine(yflat, s2):
    return pl.pallas_call(
        _combine_body,
        out_shape=jax.ShapeDtypeStruct((T_PER, D), jnp.float32),
        in_specs=[
            pl.BlockSpec(memory_space=pltpu.VMEM),
            pl.BlockSpec(memory_space=pltpu.VMEM),
        ],
        out_specs=pl.BlockSpec(memory_space=pltpu.VMEM),
        scratch_shapes=[
            pltpu.VMEM((T_PER, D), jnp.float32),
            pltpu.VMEM((T_PER, D), jnp.bfloat16),
            pltpu.VMEM((T_PER, D), jnp.bfloat16),
            pltpu.VMEM((T_PER, D), jnp.bfloat16),
            pltpu.VMEM((T_PER, D), jnp.bfloat16),
            pltpu.SemaphoreType.DMA((3,)),
            pltpu.SemaphoreType.DMA((3,)),
        ],
        compiler_params=pltpu.CompilerParams(collective_id=1),
    )(yflat, s2)


def kernel(x, router, W1, W2):
    my_y = lax.axis_index("y")

    rfull = _rexch(router.T)
    router_full = jnp.concatenate([rfull[0], rfull[1]], axis=0).T
    gates = jnp.dot(x, router_full, precision=lax.Precision.HIGHEST)
    vals, idx = lax.top_k(gates, K)
    w = jax.nn.softmax(vals, axis=-1)

    xfull, idxfull, wfull = _xexch(
        x.astype(jnp.bfloat16), idx.T, w.T.astype(jnp.float32)
    )
    xf = xfull.reshape(2 * T_PER, D)
    idx_g = jnp.concatenate([idxfull[0].T, idxfull[1].T], axis=0)
    w_g = jnp.concatenate([wfull[0].T, wfull[1].T], axis=0)

    base = my_y * E_LOCAL
    ee = idx_g.reshape(-1)
    ww = w_g.reshape(-1)
    le = ee - base
    is_local = (le >= 0) & (le < E_LOCAL)
    lec = jnp.clip(le, 0, E_LOCAL - 1)
    onehot = (lec[:, None] == jnp.arange(E_LOCAL)[None, :]) & is_local[:, None]
    pos = jnp.cumsum(onehot.astype(jnp.int32), axis=0) - 1
    pos_a = jnp.sum(jnp.where(onehot, pos, 0), axis=1)
    valid = is_local & (pos_a < C)
    slot = jnp.where(valid, lec * C + pos_a, N_SLOTS)
    s2 = slot.reshape(2 * T_PER, K).T

    fp = (lax.axis_index("x") * 2 + lax.axis_index("z")).astype(jnp.int32)
    yg = _ffn(fp[None], xf, slot[None, :], ww[None, :], W1, W2)

    return _combine(yg.reshape(N_SLOTS, D), s2)


# device time: 155505 ns/iter; 1.9136x vs baseline; 1.1650x over previous
import jax
import jax.numpy as jnp
from jax import lax
from jax.experimental import pallas as pl
from jax.experimental.pallas import tpu as pltpu

T_PER = 1024
D = 1024
F = 4096
E_LOCAL = 8
K = 2
C = 320
N_SLOTS = E_LOCAL * C
FB = 512
NF = F // FB


def _peer():
    return (lax.axis_index("x"), 1 - lax.axis_index("y"), lax.axis_index("z"))


def _neighbor_barrier(peer):
    barrier_sem = pltpu.get_barrier_semaphore()
    pl.semaphore_signal(
        barrier_sem, inc=1, device_id=peer, device_id_type=pl.DeviceIdType.MESH
    )
    pl.semaphore_wait(barrier_sem, 1)


def _rexch_body(r_ref, rfull_ref, send_sem, recv_sem):
    my_y = lax.axis_index("y")
    peer = _peer()
    _neighbor_barrier(peer)

    rfull_ref[pl.ds(my_y, 1)] = r_ref[...][None]
    rdma = pltpu.make_async_remote_copy(
        src_ref=rfull_ref.at[my_y],
        dst_ref=rfull_ref.at[my_y],
        send_sem=send_sem,
        recv_sem=recv_sem,
        device_id=peer,
        device_id_type=pl.DeviceIdType.MESH,
    )
    rdma.start()
    rdma.wait()


def _rexch(rt):
    return pl.pallas_call(
        _rexch_body,
        out_shape=jax.ShapeDtypeStruct((2, E_LOCAL, D), jnp.float32),
        in_specs=[pl.BlockSpec(memory_space=pltpu.VMEM)],
        out_specs=pl.BlockSpec(memory_space=pltpu.VMEM),
        scratch_shapes=[pltpu.SemaphoreType.DMA, pltpu.SemaphoreType.DMA],
        compiler_params=pltpu.CompilerParams(collective_id=0),
    )(rt)


def _xexch_body(
    xb_ref, idx_ref, w_ref, xfull_ref, idxfull_ref, wfull_ref,
    send_sems, recv_sems
):
    my_y = lax.axis_index("y")
    peer = _peer()
    _neighbor_barrier(peer)

    xfull_ref[pl.ds(my_y, 1)] = xb_ref[...][None]
    idxfull_ref[pl.ds(my_y, 1)] = idx_ref[...][None]
    wfull_ref[pl.ds(my_y, 1)] = w_ref[...][None]

    rdmas = [
        pltpu.make_async_remote_copy(
            src_ref=ref.at[my_y],
            dst_ref=ref.at[my_y],
            send_sem=send_sems.at[i],
            recv_sem=recv_sems.at[i],
            device_id=peer,
            device_id_type=pl.DeviceIdType.MESH,
        )
        for i, ref in enumerate([xfull_ref, idxfull_ref, wfull_ref])
    ]
    for r in rdmas:
        r.start()
    for r in rdmas:
        r.wait()


def _xexch(xb, idx, w):
    return pl.pallas_call(
        _xexch_body,
        out_shape=(
            jax.ShapeDtypeStruct((2, T_PER, D), jnp.bfloat16),
            jax.ShapeDtypeStruct((2, K, T_PER), jnp.int32),
            jax.ShapeDtypeStruct((2, K, T_PER), jnp.float32),
        ),
        in_specs=[pl.BlockSpec(memory_space=pltpu.VMEM)] * 3,
        out_specs=(pl.BlockSpec(memory_space=pltpu.VMEM),) * 3,
        scratch_shapes=[
            pltpu.SemaphoreType.DMA((3,)),
            pltpu.SemaphoreType.DMA((3,)),
        ],
        compiler_params=pltpu.CompilerParams(collective_id=2),
    )(xb, idx, w)


A = 2 * T_PER * K

N_FP = 4
NF_PER = NF // N_FP


def _ffn_body(fp_ref, xf_ref, slot_ref, ww_ref, w1_ref, w2_ref, out_ref,
              xg_ref, acc_ref, wg_ref):
    e = pl.program_id(0)
    f = pl.program_id(1)

    @pl.when(f == 0)
    def _():
        sl = slot_ref[0]
        my_slots = lax.broadcasted_iota(jnp.int32, (C, A), 0) + e * C
        ohs = my_slots == sl[None, :]
        ta = lax.broadcasted_iota(jnp.int32, (C, A), 1) // K
        tok = jnp.sum(jnp.where(ohs, ta, 0), axis=1)
        wg_ref[0, :] = jnp.sum(
            jnp.where(ohs, ww_ref[0][None, :], 0.0), axis=1
        )
        oh = (
            tok[:, None]
            == lax.broadcasted_iota(jnp.int32, (C, 2 * T_PER), 1)
        ).astype(jnp.bfloat16)
        xg_ref[...] = jnp.dot(
            oh, xf_ref[...], preferred_element_type=jnp.float32
        ).astype(jnp.bfloat16)
        acc_ref[...] = jnp.zeros_like(acc_ref)

    h = jnp.dot(
        xg_ref[...],
        w1_ref[0].astype(jnp.bfloat16),
        preferred_element_type=jnp.float32,
    )
    h = jnp.maximum(h, 0.0).astype(jnp.bfloat16)
    acc_ref[...] += jnp.dot(
        h, w2_ref[0].astype(jnp.bfloat16), preferred_element_type=jnp.float32
    )

    @pl.when(f == NF_PER - 1)
    def _():
        out_ref[0] = (acc_ref[...] * wg_ref[0, :][:, None]).astype(jnp.bfloat16)


def _ffn(fp, xf_bf, slot, ww, W1, W2):
    grid_spec = pltpu.PrefetchScalarGridSpec(
        num_scalar_prefetch=1,
        grid=(E_LOCAL, NF_PER),
        in_specs=[
            pl.BlockSpec((2 * T_PER, D), lambda e, f, fp: (0, 0)),
            pl.BlockSpec((1, A), lambda e, f, fp: (0, 0)),
            pl.BlockSpec((1, A), lambda e, f, fp: (0, 0)),
            pl.BlockSpec((1, D, FB), lambda e, f, fp: (e, 0, fp[0] * NF_PER + f)),
            pl.BlockSpec((1, FB, D), lambda e, f, fp: (e, fp[0] * NF_PER + f, 0)),
        ],
        out_specs=pl.BlockSpec((1, C, D), lambda e, f, fp: (e, 0, 0)),
        scratch_shapes=[
            pltpu.VMEM((C, D), jnp.bfloat16),
            pltpu.VMEM((C, D), jnp.float32),
            pltpu.VMEM((1, C), jnp.float32),
        ],
    )
    return pl.pallas_call(
        _ffn_body,
        grid_spec=grid_spec,
        out_shape=jax.ShapeDtypeStruct((E_LOCAL, C, D), jnp.bfloat16),
    )(fp, xf_bf, slot, ww, W1, W2)


HALF = T_PER // 2


def _undispatch_half(s2_ref, y_ref, base):
    st0 = s2_ref[0, pl.ds(base, HALF)]
    st1 = s2_ref[1, pl.ds(base, HALF)]
    iota = lax.broadcasted_iota(jnp.int32, (HALF, N_SLOTS), 1)
    oh2 = (st0[:, None] == iota).astype(jnp.bfloat16) + (
        st1[:, None] == iota
    ).astype(jnp.bfloat16)
    return jnp.dot(oh2, y_ref[...], preferred_element_type=jnp.float32)


def _combine_body(
    y_ref, s2_ref, out_ref, acc_ref, send_y, send_x, send_z,
    rbuf_y, rbuf_x, rbuf_z, send_sems, recv_sems
):
    my_x = lax.axis_index("x")
    my_y = lax.axis_index("y")
    my_z = lax.axis_index("z")
    n_y = (my_x, 1 - my_y, my_z)
    n_x = (1 - my_x, my_y, my_z)
    n_z = (my_x, my_y, 1 - my_z)

    barrier_sem = pltpu.get_barrier_semaphore()
    for nbr in (n_y, n_x, n_z):
        pl.semaphore_signal(
            barrier_sem, inc=1, device_id=nbr,
            device_id_type=pl.DeviceIdType.MESH,
        )
    pl.semaphore_wait(barrier_sem, 3)

    def exchange(sem_i, src, dst, h, nbr):
        return pltpu.make_async_remote_copy(
            src_ref=src.at[pl.ds(h * HALF, HALF)],
            dst_ref=dst.at[pl.ds(h * HALF, HALF)],
            send_sem=send_sems.at[sem_i],
            recv_sem=recv_sems.at[sem_i],
            device_id=nbr,
            device_id_type=pl.DeviceIdType.MESH,
        )

    rd_y, rd_x, rd_z = [None, None], [None, None], [None, None]

    for h in range(2):
        send_y[pl.ds(h * HALF, HALF), :] = _undispatch_half(
            s2_ref, y_ref, (1 - my_y) * T_PER + h * HALF
        ).astype(jnp.bfloat16)
        rd_y[h] = exchange(h, send_y, rbuf_y, h, n_y)
        rd_y[h].start()

    for h in range(2):
        acc_ref[pl.ds(h * HALF, HALF), :] = _undispatch_half(
            s2_ref, y_ref, my_y * T_PER + h * HALF
        )

    for h in range(2):
        rd_y[h].wait()
        rows = pl.ds(h * HALF, HALF)
        acc_ref[rows, :] += rbuf_y[rows, :].astype(jnp.float32)
        send_x[rows, :] = acc_ref[rows, :].astype(jnp.bfloat16)
        rd_x[h] = exchange(2 + h, send_x, rbuf_x, h, n_x)
        rd_x[h].start()

    for h in range(2):
        rd_x[h].wait()
        rows = pl.ds(h * HALF, HALF)
        acc_ref[rows, :] += rbuf_x[rows, :].astype(jnp.float32)
        send_z[rows, :] = acc_ref[rows, :].astype(jnp.bfloat16)
        rd_z[h] = exchange(4 + h, send_z, rbuf_z, h, n_z)
        rd_z[h].start()

    for h in range(2):
        rd_z[h].wait()
        rows = pl.ds(h * HALF, HALF)
        out_ref[rows, :] = acc_ref[rows, :] + rbuf_z[rows, :].astype(
            jnp.float32
        )


def _combine(yflat, s2):
    return pl.pallas_call(
        _combine_body,
        out_shape=jax.ShapeDtypeStruct((T_PER, D), jnp.float32),
        in_specs=[
            pl.BlockSpec(memory_space=pltpu.VMEM),
            pl.BlockSpec(memory_space=pltpu.VMEM),
        ],
        out_specs=pl.BlockSpec(memory_space=pltpu.VMEM),
        scratch_shapes=[
            pltpu.VMEM((T_PER, D), jnp.float32),
            pltpu.VMEM((T_PER, D), jnp.bfloat16),
            pltpu.VMEM((T_PER, D), jnp.bfloat16),
            pltpu.VMEM((T_PER, D), jnp.bfloat16),
            pltpu.VMEM((T_PER, D), jnp.bfloat16),
            pltpu.VMEM((T_PER, D), jnp.bfloat16),
            pltpu.VMEM((T_PER, D), jnp.bfloat16),
            pltpu.SemaphoreType.DMA((6,)),
            pltpu.SemaphoreType.DMA((6,)),
        ],
        compiler_params=pltpu.CompilerParams(collective_id=1),
    )(yflat, s2)


def kernel(x, router, W1, W2):
    my_y = lax.axis_index("y")

    rfull = _rexch(router.T)
    router_full = jnp.concatenate([rfull[0], rfull[1]], axis=0).T
    gates = jnp.dot(x, router_full, precision=lax.Precision.HIGHEST)
    vals, idx = lax.top_k(gates, K)
    w = jax.nn.softmax(vals, axis=-1)

    xfull, idxfull, wfull = _xexch(
        x.astype(jnp.bfloat16), idx.T, w.T.astype(jnp.float32)
    )
    xf = xfull.reshape(2 * T_PER, D)
    idx_g = jnp.concatenate([idxfull[0].T, idxfull[1].T], axis=0)
    w_g = jnp.concatenate([wfull[0].T, wfull[1].T], axis=0)

    base = my_y * E_LOCAL
    ee = idx_g.reshape(-1)
    ww = w_g.reshape(-1)
    le = ee - base
    is_local = (le >= 0) & (le < E_LOCAL)
    lec = jnp.clip(le, 0, E_LOCAL - 1)
    onehot = (lec[:, None] == jnp.arange(E_LOCAL)[None, :]) & is_local[:, None]
    pos = jnp.cumsum(onehot.astype(jnp.int32), axis=0) - 1
    pos_a = jnp.sum(jnp.where(onehot, pos, 0), axis=1)
    valid = is_local & (pos_a < C)
    slot = jnp.where(valid, lec * C + pos_a, N_SLOTS)
    s2 = slot.reshape(2 * T_PER, K).T

    fp = (lax.axis_index("x") * 2 + lax.axis_index("z")).astype(jnp.int32)
    yg = _ffn(fp[None], xf, slot[None, :], ww[None, :], W1, W2)

    return _combine(yg.reshape(N_SLOTS, D), s2)


# device time: 154498 ns/iter; 1.9261x vs baseline; 1.0065x over previous
import jax
import jax.numpy as jnp
from jax import lax
from jax.experimental import pallas as pl
from jax.experimental.pallas import tpu as pltpu

T_PER = 1024
D = 1024
F = 4096
E_LOCAL = 8
K = 2
C = 320
N_SLOTS = E_LOCAL * C
FB = 512
NF = F // FB


def _peer():
    return (lax.axis_index("x"), 1 - lax.axis_index("y"), lax.axis_index("z"))


def _neighbor_barrier(peer):
    barrier_sem = pltpu.get_barrier_semaphore()
    pl.semaphore_signal(
        barrier_sem, inc=1, device_id=peer, device_id_type=pl.DeviceIdType.MESH
    )
    pl.semaphore_wait(barrier_sem, 1)


N_E = 2 * E_LOCAL


def _gate_exch_body(
    x_ref, rt_ref, xfull_ref, idxfull_ref, wfull_ref, rfull_ref,
    send_sems, recv_sems
):
    my_y = lax.axis_index("y")
    peer = _peer()
    _neighbor_barrier(peer)

    def exch(i, ref):
        return pltpu.make_async_remote_copy(
            src_ref=ref.at[my_y],
            dst_ref=ref.at[my_y],
            send_sem=send_sems.at[i],
            recv_sem=recv_sems.at[i],
            device_id=peer,
            device_id_type=pl.DeviceIdType.MESH,
        )

    xfull_ref[pl.ds(my_y, 1)] = x_ref[...].astype(jnp.bfloat16)[None]
    rd_x = exch(0, xfull_ref)
    rd_x.start()

    rfull_ref[pl.ds(my_y, 1)] = rt_ref[...][None]
    rd_r = exch(1, rfull_ref)
    rd_r.start()
    rd_r.wait()

    rcat = jnp.concatenate([rfull_ref[0], rfull_ref[1]], axis=0)
    g = lax.dot_general(
        x_ref[...], rcat, (((1,), (1,)), ((), ())),
        preferred_element_type=jnp.float32,
        precision=lax.Precision.HIGHEST,
    )
    iota16 = lax.broadcasted_iota(jnp.int32, (T_PER, N_E), 1)
    m1 = jnp.max(g, axis=1)
    i1 = jnp.argmax(g, axis=1).astype(jnp.int32)
    g2 = jnp.where(iota16 == i1[:, None], -jnp.inf, g)
    m2 = jnp.max(g2, axis=1)
    i2 = jnp.argmax(g2, axis=1).astype(jnp.int32)
    w1 = 1.0 / (1.0 + jnp.exp(m2 - m1))

    idxfull_ref[pl.ds(my_y, 1)] = jnp.concatenate(
        [i1[None, None, :], i2[None, None, :]], axis=1
    )
    wfull_ref[pl.ds(my_y, 1)] = jnp.concatenate(
        [w1[None, None, :], (1.0 - w1)[None, None, :]], axis=1
    )
    rd_i = exch(2, idxfull_ref)
    rd_w = exch(3, wfull_ref)
    rd_i.start()
    rd_w.start()
    rd_i.wait()
    rd_w.wait()
    rd_x.wait()


def _gate_exch(x, rt):
    return pl.pallas_call(
        _gate_exch_body,
        out_shape=(
            jax.ShapeDtypeStruct((2, T_PER, D), jnp.bfloat16),
            jax.ShapeDtypeStruct((2, K, T_PER), jnp.int32),
            jax.ShapeDtypeStruct((2, K, T_PER), jnp.float32),
        ),
        in_specs=[pl.BlockSpec(memory_space=pltpu.VMEM)] * 2,
        out_specs=(pl.BlockSpec(memory_space=pltpu.VMEM),) * 3,
        scratch_shapes=[
            pltpu.VMEM((2, E_LOCAL, D), jnp.float32),
            pltpu.SemaphoreType.DMA((4,)),
            pltpu.SemaphoreType.DMA((4,)),
        ],
        compiler_params=pltpu.CompilerParams(collective_id=0),
    )(x, rt)


A = 2 * T_PER * K

N_FP = 4
NF_PER = NF // N_FP


def _ffn_body(fp_ref, xf_ref, slot_ref, ww_ref, w1_ref, w2_ref, out_ref,
              xg_ref, acc_ref, wg_ref):
    e = pl.program_id(0)
    f = pl.program_id(1)

    @pl.when(f == 0)
    def _():
        sl = slot_ref[0]
        my_slots = lax.broadcasted_iota(jnp.int32, (C, A), 0) + e * C
        ohs = my_slots == sl[None, :]
        ta = lax.broadcasted_iota(jnp.int32, (C, A), 1) // K
        tok = jnp.sum(jnp.where(ohs, ta, 0), axis=1)
        wg_ref[0, :] = jnp.sum(
            jnp.where(ohs, ww_ref[0][None, :], 0.0), axis=1
        )
        oh = (
            tok[:, None]
            == lax.broadcasted_iota(jnp.int32, (C, 2 * T_PER), 1)
        ).astype(jnp.bfloat16)
        xg_ref[...] = jnp.dot(
            oh, xf_ref[...], preferred_element_type=jnp.float32
        ).astype(jnp.bfloat16)
        acc_ref[...] = jnp.zeros_like(acc_ref)

    h = jnp.dot(
        xg_ref[...],
        w1_ref[0].astype(jnp.bfloat16),
        preferred_element_type=jnp.float32,
    )
    h = jnp.maximum(h, 0.0).astype(jnp.bfloat16)
    acc_ref[...] += jnp.dot(
        h, w2_ref[0].astype(jnp.bfloat16), preferred_element_type=jnp.float32
    )

    @pl.when(f == NF_PER - 1)
    def _():
        out_ref[0] = (acc_ref[...] * wg_ref[0, :][:, None]).astype(jnp.bfloat16)


def _ffn(fp, xf_bf, slot, ww, W1, W2):
    grid_spec = pltpu.PrefetchScalarGridSpec(
        num_scalar_prefetch=1,
        grid=(E_LOCAL, NF_PER),
        in_specs=[
            pl.BlockSpec((2 * T_PER, D), lambda e, f, fp: (0, 0)),
            pl.BlockSpec((1, A), lambda e, f, fp: (0, 0)),
            pl.BlockSpec((1, A), lambda e, f, fp: (0, 0)),
            pl.BlockSpec((1, D, FB), lambda e, f, fp: (e, 0, fp[0] * NF_PER + f)),
            pl.BlockSpec((1, FB, D), lambda e, f, fp: (e, fp[0] * NF_PER + f, 0)),
        ],
        out_specs=pl.BlockSpec((1, C, D), lambda e, f, fp: (e, 0, 0)),
        scratch_shapes=[
            pltpu.VMEM((C, D), jnp.bfloat16),
            pltpu.VMEM((C, D), jnp.float32),
            pltpu.VMEM((1, C), jnp.float32),
        ],
    )
    return pl.pallas_call(
        _ffn_body,
        grid_spec=grid_spec,
        out_shape=jax.ShapeDtypeStruct((E_LOCAL, C, D), jnp.bfloat16),
    )(fp, xf_bf, slot, ww, W1, W2)


HALF = T_PER // 2


def _undispatch_half(s2_ref, y_ref, base):
    st0 = s2_ref[0, pl.ds(base, HALF)]
    st1 = s2_ref[1, pl.ds(base, HALF)]
    iota = lax.broadcasted_iota(jnp.int32, (HALF, N_SLOTS), 1)
    oh2 = (st0[:, None] == iota).astype(jnp.bfloat16) + (
        st1[:, None] == iota
    ).astype(jnp.bfloat16)
    return jnp.dot(oh2, y_ref[...], preferred_element_type=jnp.float32)


def _combine_body(
    y_ref, s2_ref, out_ref, acc_ref, send_y, send_x, send_z,
    rbuf_y, rbuf_x, rbuf_z, send_sems, recv_sems
):
    my_x = lax.axis_index("x")
    my_y = lax.axis_index("y")
    my_z = lax.axis_index("z")
    n_y = (my_x, 1 - my_y, my_z)
    n_x = (1 - my_x, my_y, my_z)
    n_z = (my_x, my_y, 1 - my_z)

    barrier_sem = pltpu.get_barrier_semaphore()
    for nbr in (n_y, n_x, n_z):
        pl.semaphore_signal(
            barrier_sem, inc=1, device_id=nbr,
            device_id_type=pl.DeviceIdType.MESH,
        )
    pl.semaphore_wait(barrier_sem, 3)

    def exchange(sem_i, src, dst, h, nbr):
        return pltpu.make_async_remote_copy(
            src_ref=src.at[pl.ds(h * HALF, HALF)],
            dst_ref=dst.at[pl.ds(h * HALF, HALF)],
            send_sem=send_sems.at[sem_i],
            recv_sem=recv_sems.at[sem_i],
            device_id=nbr,
            device_id_type=pl.DeviceIdType.MESH,
        )

    rd_y, rd_x, rd_z = [None, None], [None, None], [None, None]

    for h in range(2):
        send_y[pl.ds(h * HALF, HALF), :] = _undispatch_half(
            s2_ref, y_ref, (1 - my_y) * T_PER + h * HALF
        ).astype(jnp.bfloat16)
        rd_y[h] = exchange(h, send_y, rbuf_y, h, n_y)
        rd_y[h].start()

    for h in range(2):
        acc_ref[pl.ds(h * HALF, HALF), :] = _undispatch_half(
            s2_ref, y_ref, my_y * T_PER + h * HALF
        )

    for h in range(2):
        rd_y[h].wait()
        rows = pl.ds(h * HALF, HALF)
        acc_ref[rows, :] += rbuf_y[rows, :].astype(jnp.float32)
        send_x[rows, :] = acc_ref[rows, :].astype(jnp.bfloat16)
        rd_x[h] = exchange(2 + h, send_x, rbuf_x, h, n_x)
        rd_x[h].start()

    for h in range(2):
        rd_x[h].wait()
        rows = pl.ds(h * HALF, HALF)
        acc_ref[rows, :] += rbuf_x[rows, :].astype(jnp.float32)
        send_z[rows, :] = acc_ref[rows, :].astype(jnp.bfloat16)
        rd_z[h] = exchange(4 + h, send_z, rbuf_z, h, n_z)
        rd_z[h].start()

    for h in range(2):
        rd_z[h].wait()
        rows = pl.ds(h * HALF, HALF)
        out_ref[rows, :] = acc_ref[rows, :] + rbuf_z[rows, :].astype(
            jnp.float32
        )


def _combine(yflat, s2):
    return pl.pallas_call(
        _combine_body,
        out_shape=jax.ShapeDtypeStruct((T_PER, D), jnp.float32),
        in_specs=[
            pl.BlockSpec(memory_space=pltpu.VMEM),
            pl.BlockSpec(memory_space=pltpu.VMEM),
        ],
        out_specs=pl.BlockSpec(memory_space=pltpu.VMEM),
        scratch_shapes=[
            pltpu.VMEM((T_PER, D), jnp.float32),
            pltpu.VMEM((T_PER, D), jnp.bfloat16),
            pltpu.VMEM((T_PER, D), jnp.bfloat16),
            pltpu.VMEM((T_PER, D), jnp.bfloat16),
            pltpu.VMEM((T_PER, D), jnp.bfloat16),
            pltpu.VMEM((T_PER, D), jnp.bfloat16),
            pltpu.VMEM((T_PER, D), jnp.bfloat16),
            pltpu.SemaphoreType.DMA((6,)),
            pltpu.SemaphoreType.DMA((6,)),
        ],
        compiler_params=pltpu.CompilerParams(collective_id=1),
    )(yflat, s2)


def kernel(x, router, W1, W2):
    my_y = lax.axis_index("y")

    xfull, idxfull, wfull = _gate_exch(x, router.T)
    xf = xfull.reshape(2 * T_PER, D)
    idx_g = jnp.concatenate([idxfull[0].T, idxfull[1].T], axis=0)
    w_g = jnp.concatenate([wfull[0].T, wfull[1].T], axis=0)

    base = my_y * E_LOCAL
    ee = idx_g.reshape(-1)
    ww = w_g.reshape(-1)
    le = ee - base
    is_local = (le >= 0) & (le < E_LOCAL)
    lec = jnp.clip(le, 0, E_LOCAL - 1)
    onehot = (lec[:, None] == jnp.arange(E_LOCAL)[None, :]) & is_local[:, None]
    pos = jnp.cumsum(onehot.astype(jnp.int32), axis=0) - 1
    pos_a = jnp.sum(jnp.where(onehot, pos, 0), axis=1)
    valid = is_local & (pos_a < C)
    slot = jnp.where(valid, lec * C + pos_a, N_SLOTS)
    s2 = slot.reshape(2 * T_PER, K).T

    fp = (lax.axis_index("x") * 2 + lax.axis_index("z")).astype(jnp.int32)
    yg = _ffn(fp[None], xf, slot[None, :], ww[None, :], W1, W2)

    return _combine(yg.reshape(N_SLOTS, D), s2)


# device time: 150921 ns/iter; 1.9718x vs baseline; 1.0237x over previous
import jax
import jax.numpy as jnp
from jax import lax
from jax.experimental import pallas as pl
from jax.experimental.pallas import tpu as pltpu

T_PER = 1024
D = 1024
F = 4096
E_LOCAL = 8
K = 2
C = 320
N_SLOTS = E_LOCAL * C
FB = 512
NF = F // FB


def _peer():
    return (lax.axis_index("x"), 1 - lax.axis_index("y"), lax.axis_index("z"))


def _neighbor_barrier(peer):
    barrier_sem = pltpu.get_barrier_semaphore()
    pl.semaphore_signal(
        barrier_sem, inc=1, device_id=peer, device_id_type=pl.DeviceIdType.MESH
    )
    pl.semaphore_wait(barrier_sem, 1)


N_E = 2 * E_LOCAL


def _gate_exch_body(
    x_ref, rt_ref, xfull_ref, idxfull_ref, wfull_ref, rfull_ref,
    send_sems, recv_sems
):
    my_y = lax.axis_index("y")
    peer = _peer()
    _neighbor_barrier(peer)

    def exch(i, ref):
        return pltpu.make_async_remote_copy(
            src_ref=ref.at[my_y],
            dst_ref=ref.at[my_y],
            send_sem=send_sems.at[i],
            recv_sem=recv_sems.at[i],
            device_id=peer,
            device_id_type=pl.DeviceIdType.MESH,
        )

    xfull_ref[pl.ds(my_y, 1)] = x_ref[...].astype(jnp.bfloat16)[None]
    rd_x = exch(0, xfull_ref)
    rd_x.start()

    rfull_ref[pl.ds(my_y, 1)] = rt_ref[...][None]
    rd_r = exch(1, rfull_ref)
    rd_r.start()
    rd_r.wait()

    rcat = jnp.concatenate([rfull_ref[0], rfull_ref[1]], axis=0)
    g = lax.dot_general(
        x_ref[...], rcat, (((1,), (1,)), ((), ())),
        preferred_element_type=jnp.float32,
        precision=lax.Precision.HIGHEST,
    )
    iota16 = lax.broadcasted_iota(jnp.int32, (T_PER, N_E), 1)
    m1 = jnp.max(g, axis=1)
    i1 = jnp.argmax(g, axis=1).astype(jnp.int32)
    g2 = jnp.where(iota16 == i1[:, None], -jnp.inf, g)
    m2 = jnp.max(g2, axis=1)
    i2 = jnp.argmax(g2, axis=1).astype(jnp.int32)
    w1 = 1.0 / (1.0 + jnp.exp(m2 - m1))

    idxfull_ref[pl.ds(my_y, 1)] = jnp.concatenate(
        [i1[None, None, :], i2[None, None, :]], axis=1
    )
    wfull_ref[pl.ds(my_y, 1)] = jnp.concatenate(
        [w1[None, None, :], (1.0 - w1)[None, None, :]], axis=1
    )
    rd_i = exch(2, idxfull_ref)
    rd_w = exch(3, wfull_ref)
    rd_i.start()
    rd_w.start()
    rd_i.wait()
    rd_w.wait()
    rd_x.wait()


def _gate_exch(x, rt):
    return pl.pallas_call(
        _gate_exch_body,
        out_shape=(
            jax.ShapeDtypeStruct((2, T_PER, D), jnp.bfloat16),
            jax.ShapeDtypeStruct((2, K, T_PER), jnp.int32),
            jax.ShapeDtypeStruct((2, K, T_PER), jnp.float32),
        ),
        in_specs=[pl.BlockSpec(memory_space=pltpu.VMEM)] * 2,
        out_specs=(pl.BlockSpec(memory_space=pltpu.VMEM),) * 3,
        scratch_shapes=[
            pltpu.VMEM((2, E_LOCAL, D), jnp.float32),
            pltpu.SemaphoreType.DMA((4,)),
            pltpu.SemaphoreType.DMA((4,)),
        ],
        compiler_params=pltpu.CompilerParams(collective_id=0),
    )(x, rt)


A = 2 * T_PER * K

N_FP = 4
NF_PER = NF // N_FP


def _ffn_body(fp_ref, xf_ref, slot_ref, ww_ref, w1_ref, w2_ref, out_ref,
              xg_ref, acc_ref, wg_ref):
    e = pl.program_id(0)
    f = pl.program_id(1)

    @pl.when(f == 0)
    def _():
        sl = slot_ref[0]
        my_slots = lax.broadcasted_iota(jnp.int32, (C, A), 0) + e * C
        ohs = my_slots == sl[None, :]
        ta = lax.broadcasted_iota(jnp.int32, (C, A), 1) // K
        tok = jnp.sum(jnp.where(ohs, ta, 0), axis=1)
        wg_ref[0, :] = jnp.sum(
            jnp.where(ohs, ww_ref[0][None, :], 0.0), axis=1
        )
        oh = (
            tok[:, None]
            == lax.broadcasted_iota(jnp.int32, (C, 2 * T_PER), 1)
        ).astype(jnp.bfloat16)
        xg_ref[...] = jnp.dot(
            oh, xf_ref[...], preferred_element_type=jnp.float32
        ).astype(jnp.bfloat16)
        acc_ref[...] = jnp.zeros_like(acc_ref)

    h = jnp.dot(
        xg_ref[...],
        w1_ref[0].astype(jnp.bfloat16),
        preferred_element_type=jnp.float32,
    )
    h = jnp.maximum(h, 0.0).astype(jnp.bfloat16)
    acc_ref[...] += jnp.dot(
        h, w2_ref[0].astype(jnp.bfloat16), preferred_element_type=jnp.float32
    )

    @pl.when(f == NF_PER - 1)
    def _():
        out_ref[0] = (acc_ref[...] * wg_ref[0, :][:, None]).astype(jnp.bfloat16)


def _ffn(fp, xf_bf, slot, ww, W1, W2):
    grid_spec = pltpu.PrefetchScalarGridSpec(
        num_scalar_prefetch=1,
        grid=(E_LOCAL, NF_PER),
        in_specs=[
            pl.BlockSpec((2 * T_PER, D), lambda e, f, fp: (0, 0)),
            pl.BlockSpec((1, A), lambda e, f, fp: (0, 0)),
            pl.BlockSpec((1, A), lambda e, f, fp: (0, 0)),
            pl.BlockSpec((1, D, FB), lambda e, f, fp: (e, 0, fp[0] * NF_PER + f)),
            pl.BlockSpec((1, FB, D), lambda e, f, fp: (e, fp[0] * NF_PER + f, 0)),
        ],
        out_specs=pl.BlockSpec((1, C, D), lambda e, f, fp: (e, 0, 0)),
        scratch_shapes=[
            pltpu.VMEM((C, D), jnp.bfloat16),
            pltpu.VMEM((C, D), jnp.float32),
            pltpu.VMEM((1, C), jnp.float32),
        ],
    )
    return pl.pallas_call(
        _ffn_body,
        grid_spec=grid_spec,
        out_shape=jax.ShapeDtypeStruct((E_LOCAL, C, D), jnp.bfloat16),
    )(fp, xf_bf, slot, ww, W1, W2)


N_PIPE = 4
SEG = T_PER // N_PIPE


def _undispatch_seg(s2_ref, y_ref, base):
    st0 = s2_ref[0, pl.ds(base, SEG)]
    st1 = s2_ref[1, pl.ds(base, SEG)]
    iota = lax.broadcasted_iota(jnp.int32, (SEG, N_SLOTS), 1)
    oh2 = (st0[:, None] == iota).astype(jnp.bfloat16) + (
        st1[:, None] == iota
    ).astype(jnp.bfloat16)
    return jnp.dot(oh2, y_ref[...], preferred_element_type=jnp.float32)


def _combine_body(
    y_ref, s2_ref, out_ref, acc_ref, send_y, send_x, send_z,
    rbuf_y, rbuf_x, rbuf_z, send_sems, recv_sems
):
    my_x = lax.axis_index("x")
    my_y = lax.axis_index("y")
    my_z = lax.axis_index("z")
    n_y = (my_x, 1 - my_y, my_z)
    n_x = (1 - my_x, my_y, my_z)
    n_z = (my_x, my_y, 1 - my_z)

    barrier_sem = pltpu.get_barrier_semaphore()
    for nbr in (n_y, n_x, n_z):
        pl.semaphore_signal(
            barrier_sem, inc=1, device_id=nbr,
            device_id_type=pl.DeviceIdType.MESH,
        )
    pl.semaphore_wait(barrier_sem, 3)

    def exchange(sem_i, src, dst, q, nbr):
        return pltpu.make_async_remote_copy(
            src_ref=src.at[pl.ds(q * SEG, SEG)],
            dst_ref=dst.at[pl.ds(q * SEG, SEG)],
            send_sem=send_sems.at[sem_i],
            recv_sem=recv_sems.at[sem_i],
            device_id=nbr,
            device_id_type=pl.DeviceIdType.MESH,
        )

    rd_y = [None] * N_PIPE
    rd_x = [None] * N_PIPE
    rd_z = [None] * N_PIPE

    for q in range(N_PIPE):
        send_y[pl.ds(q * SEG, SEG), :] = _undispatch_seg(
            s2_ref, y_ref, (1 - my_y) * T_PER + q * SEG
        ).astype(jnp.bfloat16)
        rd_y[q] = exchange(q, send_y, rbuf_y, q, n_y)
        rd_y[q].start()

    for q in range(N_PIPE):
        acc_ref[pl.ds(q * SEG, SEG), :] = _undispatch_seg(
            s2_ref, y_ref, my_y * T_PER + q * SEG
        )

    for q in range(N_PIPE):
        rd_y[q].wait()
        rows = pl.ds(q * SEG, SEG)
        acc_ref[rows, :] += rbuf_y[rows, :].astype(jnp.float32)
        send_x[rows, :] = acc_ref[rows, :].astype(jnp.bfloat16)
        rd_x[q] = exchange(N_PIPE + q, send_x, rbuf_x, q, n_x)
        rd_x[q].start()

    for q in range(N_PIPE):
        rd_x[q].wait()
        rows = pl.ds(q * SEG, SEG)
        acc_ref[rows, :] += rbuf_x[rows, :].astype(jnp.float32)
        send_z[rows, :] = acc_ref[rows, :].astype(jnp.bfloat16)
        rd_z[q] = exchange(2 * N_PIPE + q, send_z, rbuf_z, q, n_z)
        rd_z[q].start()

    for q in range(N_PIPE):
        rd_z[q].wait()
        rows = pl.ds(q * SEG, SEG)
        out_ref[rows, :] = acc_ref[rows, :] + rbuf_z[rows, :].astype(
            jnp.float32
        )


def _combine(yflat, s2):
    return pl.pallas_call(
        _combine_body,
        out_shape=jax.ShapeDtypeStruct((T_PER, D), jnp.float32),
        in_specs=[
            pl.BlockSpec(memory_space=pltpu.VMEM),
            pl.BlockSpec(memory_space=pltpu.VMEM),
        ],
        out_specs=pl.BlockSpec(memory_space=pltpu.VMEM),
        scratch_shapes=[
            pltpu.VMEM((T_PER, D), jnp.float32),
            pltpu.VMEM((T_PER, D), jnp.bfloat16),
            pltpu.VMEM((T_PER, D), jnp.bfloat16),
            pltpu.VMEM((T_PER, D), jnp.bfloat16),
            pltpu.VMEM((T_PER, D), jnp.bfloat16),
            pltpu.VMEM((T_PER, D), jnp.bfloat16),
            pltpu.VMEM((T_PER, D), jnp.bfloat16),
            pltpu.SemaphoreType.DMA((3 * N_PIPE,)),
            pltpu.SemaphoreType.DMA((3 * N_PIPE,)),
        ],
        compiler_params=pltpu.CompilerParams(collective_id=1),
    )(yflat, s2)


def kernel(x, router, W1, W2):
    my_y = lax.axis_index("y")

    xfull, idxfull, wfull = _gate_exch(x, router.T)
    xf = xfull.reshape(2 * T_PER, D)
    idx_g = jnp.concatenate([idxfull[0].T, idxfull[1].T], axis=0)
    w_g = jnp.concatenate([wfull[0].T, wfull[1].T], axis=0)

    base = my_y * E_LOCAL
    ee = idx_g.reshape(-1)
    ww = w_g.reshape(-1)
    le = ee - base
    is_local = (le >= 0) & (le < E_LOCAL)
    lec = jnp.clip(le, 0, E_LOCAL - 1)
    onehot = (lec[:, None] == jnp.arange(E_LOCAL)[None, :]) & is_local[:, None]
    pos = jnp.cumsum(onehot.astype(jnp.int32), axis=0) - 1
    pos_a = jnp.sum(jnp.where(onehot, pos, 0), axis=1)
    valid = is_local & (pos_a < C)
    slot = jnp.where(valid, lec * C + pos_a, N_SLOTS)
    s2 = slot.reshape(2 * T_PER, K).T

    fp = (lax.axis_index("x") * 2 + lax.axis_index("z")).astype(jnp.int32)
    yg = _ffn(fp[None], xf, slot[None, :], ww[None, :], W1, W2)

    return _combine(yg.reshape(N_SLOTS, D), s2)


# device time: 141606 ns/iter; 2.1015x vs baseline; 1.0658x over previous
import jax
import jax.numpy as jnp
from jax import lax
from jax.experimental import pallas as pl
from jax.experimental.pallas import tpu as pltpu

T_PER = 1024
D = 1024
F = 4096
E_LOCAL = 8
K = 2
C = 320
N_SLOTS = E_LOCAL * C
FB = 512
NF = F // FB


def _peer():
    return (lax.axis_index("x"), 1 - lax.axis_index("y"), lax.axis_index("z"))


def _neighbor_barrier(peer):
    barrier_sem = pltpu.get_barrier_semaphore()
    pl.semaphore_signal(
        barrier_sem, inc=1, device_id=peer, device_id_type=pl.DeviceIdType.MESH
    )
    pl.semaphore_wait(barrier_sem, 1)


N_E = 2 * E_LOCAL


def _gate_exch_body(
    x_ref, rt_ref, xfull_ref, idxfull_ref, wfull_ref, rfull_ref,
    send_sems, recv_sems
):
    my_y = lax.axis_index("y")
    peer = _peer()
    _neighbor_barrier(peer)

    def exch(i, ref):
        return pltpu.make_async_remote_copy(
            src_ref=ref.at[my_y],
            dst_ref=ref.at[my_y],
            send_sem=send_sems.at[i],
            recv_sem=recv_sems.at[i],
            device_id=peer,
            device_id_type=pl.DeviceIdType.MESH,
        )

    xfull_ref[pl.ds(my_y, 1)] = x_ref[...].astype(jnp.bfloat16)[None]
    rd_x = exch(0, xfull_ref)
    rd_x.start()

    rfull_ref[pl.ds(my_y, 1)] = rt_ref[...][None]
    rd_r = exch(1, rfull_ref)
    rd_r.start()
    rd_r.wait()

    rcat = jnp.concatenate([rfull_ref[0], rfull_ref[1]], axis=0)
    g = lax.dot_general(
        x_ref[...], rcat, (((1,), (1,)), ((), ())),
        preferred_element_type=jnp.float32,
        precision=lax.Precision.HIGHEST,
    )
    iota16 = lax.broadcasted_iota(jnp.int32, (T_PER, N_E), 1)
    m1 = jnp.max(g, axis=1)
    i1 = jnp.argmax(g, axis=1).astype(jnp.int32)
    g2 = jnp.where(iota16 == i1[:, None], -jnp.inf, g)
    m2 = jnp.max(g2, axis=1)
    i2 = jnp.argmax(g2, axis=1).astype(jnp.int32)
    w1 = 1.0 / (1.0 + jnp.exp(m2 - m1))

    idxfull_ref[pl.ds(my_y, 1)] = jnp.concatenate(
        [i1[None, None, :], i2[None, None, :]], axis=1
    )
    wfull_ref[pl.ds(my_y, 1)] = jnp.concatenate(
        [w1[None, None, :], (1.0 - w1)[None, None, :]], axis=1
    )
    rd_i = exch(2, idxfull_ref)
    rd_w = exch(3, wfull_ref)
    rd_i.start()
    rd_w.start()
    rd_i.wait()
    rd_w.wait()
    rd_x.wait()


def _gate_exch(x, rt):
    return pl.pallas_call(
        _gate_exch_body,
        out_shape=(
            jax.ShapeDtypeStruct((2, T_PER, D), jnp.bfloat16),
            jax.ShapeDtypeStruct((2, K, T_PER), jnp.int32),
            jax.ShapeDtypeStruct((2, K, T_PER), jnp.float32),
        ),
        in_specs=[pl.BlockSpec(memory_space=pltpu.VMEM)] * 2,
        out_specs=(pl.BlockSpec(memory_space=pltpu.VMEM),) * 3,
        scratch_shapes=[
            pltpu.VMEM((2, E_LOCAL, D), jnp.float32),
            pltpu.SemaphoreType.DMA((4,)),
            pltpu.SemaphoreType.DMA((4,)),
        ],
        compiler_params=pltpu.CompilerParams(collective_id=0),
    )(x, rt)


A = 2 * T_PER * K

N_FP = 4
NF_PER = NF // N_FP


def _ffn_body(fp_ref, xf_ref, s2_ref, w2_ref_in, w1_ref, w2_ref, out_ref,
              xg_ref, acc_ref, wg_ref):
    e = pl.program_id(0)
    f = pl.program_id(1)

    @pl.when(f == 0)
    def _():
        iota_c = lax.broadcasted_iota(jnp.int32, (C, 2 * T_PER), 0) + e * C
        oh0 = iota_c == s2_ref[0][None, :]
        oh1 = iota_c == s2_ref[1][None, :]
        ohc = oh0.astype(jnp.bfloat16) + oh1.astype(jnp.bfloat16)
        xg_ref[...] = jnp.dot(
            ohc, xf_ref[...], preferred_element_type=jnp.float32
        ).astype(jnp.bfloat16)
        wg_ref[0, :] = jnp.sum(
            jnp.where(oh0, w2_ref_in[0][None, :], 0.0)
            + jnp.where(oh1, w2_ref_in[1][None, :], 0.0),
            axis=1,
        )
        acc_ref[...] = jnp.zeros_like(acc_ref)

    h = jnp.dot(
        xg_ref[...],
        w1_ref[0].astype(jnp.bfloat16),
        preferred_element_type=jnp.float32,
    )
    h = jnp.maximum(h, 0.0).astype(jnp.bfloat16)
    acc_ref[...] += jnp.dot(
        h, w2_ref[0].astype(jnp.bfloat16), preferred_element_type=jnp.float32
    )

    @pl.when(f == NF_PER - 1)
    def _():
        out_ref[0] = (acc_ref[...] * wg_ref[0, :][:, None]).astype(jnp.bfloat16)


def _ffn(fp, xf_bf, s2, w2, W1, W2):
    grid_spec = pltpu.PrefetchScalarGridSpec(
        num_scalar_prefetch=1,
        grid=(E_LOCAL, NF_PER),
        in_specs=[
            pl.BlockSpec((2 * T_PER, D), lambda e, f, fp: (0, 0)),
            pl.BlockSpec((K, 2 * T_PER), lambda e, f, fp: (0, 0)),
            pl.BlockSpec((K, 2 * T_PER), lambda e, f, fp: (0, 0)),
            pl.BlockSpec((1, D, FB), lambda e, f, fp: (e, 0, fp[0] * NF_PER + f)),
            pl.BlockSpec((1, FB, D), lambda e, f, fp: (e, fp[0] * NF_PER + f, 0)),
        ],
        out_specs=pl.BlockSpec((1, C, D), lambda e, f, fp: (e, 0, 0)),
        scratch_shapes=[
            pltpu.VMEM((C, D), jnp.bfloat16),
            pltpu.VMEM((C, D), jnp.float32),
            pltpu.VMEM((1, C), jnp.float32),
        ],
    )
    return pl.pallas_call(
        _ffn_body,
        grid_spec=grid_spec,
        out_shape=jax.ShapeDtypeStruct((E_LOCAL, C, D), jnp.bfloat16),
    )(fp, xf_bf, s2, w2, W1, W2)


N_PIPE = 4
SEG = T_PER // N_PIPE


def _undispatch_seg(s2_ref, y_ref, base):
    st0 = s2_ref[0, pl.ds(base, SEG)]
    st1 = s2_ref[1, pl.ds(base, SEG)]
    iota = lax.broadcasted_iota(jnp.int32, (SEG, N_SLOTS), 1)
    oh2 = (st0[:, None] == iota).astype(jnp.bfloat16) + (
        st1[:, None] == iota
    ).astype(jnp.bfloat16)
    return jnp.dot(oh2, y_ref[...], preferred_element_type=jnp.float32)


def _combine_body(
    y_ref, s2_ref, out_ref, acc_ref, send_y, send_x, send_z,
    rbuf_y, rbuf_x, rbuf_z, send_sems, recv_sems
):
    my_x = lax.axis_index("x")
    my_y = lax.axis_index("y")
    my_z = lax.axis_index("z")
    n_y = (my_x, 1 - my_y, my_z)
    n_x = (1 - my_x, my_y, my_z)
    n_z = (my_x, my_y, 1 - my_z)

    barrier_sem = pltpu.get_barrier_semaphore()
    for nbr in (n_y, n_x, n_z):
        pl.semaphore_signal(
            barrier_sem, inc=1, device_id=nbr,
            device_id_type=pl.DeviceIdType.MESH,
        )
    pl.semaphore_wait(barrier_sem, 3)

    def exchange(sem_i, src, dst, q, nbr):
        return pltpu.make_async_remote_copy(
            src_ref=src.at[pl.ds(q * SEG, SEG)],
            dst_ref=dst.at[pl.ds(q * SEG, SEG)],
            send_sem=send_sems.at[sem_i],
            recv_sem=recv_sems.at[sem_i],
            device_id=nbr,
            device_id_type=pl.DeviceIdType.MESH,
        )

    rd_y = [None] * N_PIPE
    rd_x = [None] * N_PIPE
    rd_z = [None] * N_PIPE

    for q in range(N_PIPE):
        send_y[pl.ds(q * SEG, SEG), :] = _undispatch_seg(
            s2_ref, y_ref, (1 - my_y) * T_PER + q * SEG
        ).astype(jnp.bfloat16)
        rd_y[q] = exchange(q, send_y, rbuf_y, q, n_y)
        rd_y[q].start()

    for q in range(N_PIPE):
        acc_ref[pl.ds(q * SEG, SEG), :] = _undispatch_seg(
            s2_ref, y_ref, my_y * T_PER + q * SEG
        )

    for q in range(N_PIPE):
        rd_y[q].wait()
        rows = pl.ds(q * SEG, SEG)
        acc_ref[rows, :] += rbuf_y[rows, :].astype(jnp.float32)
        send_x[rows, :] = acc_ref[rows, :].astype(jnp.bfloat16)
        rd_x[q] = exchange(N_PIPE + q, send_x, rbuf_x, q, n_x)
        rd_x[q].start()

    for q in range(N_PIPE):
        rd_x[q].wait()
        rows = pl.ds(q * SEG, SEG)
        acc_ref[rows, :] += rbuf_x[rows, :].astype(jnp.float32)
        send_z[rows, :] = acc_ref[rows, :].astype(jnp.bfloat16)
        rd_z[q] = exchange(2 * N_PIPE + q, send_z, rbuf_z, q, n_z)
        rd_z[q].start()

    for q in range(N_PIPE):
        rd_z[q].wait()
        rows = pl.ds(q * SEG, SEG)
        out_ref[rows, :] = acc_ref[rows, :] + rbuf_z[rows, :].astype(
            jnp.float32
        )


def _combine(yflat, s2):
    return pl.pallas_call(
        _combine_body,
        out_shape=jax.ShapeDtypeStruct((T_PER, D), jnp.float32),
        in_specs=[
            pl.BlockSpec(memory_space=pltpu.VMEM),
            pl.BlockSpec(memory_space=pltpu.VMEM),
        ],
        out_specs=pl.BlockSpec(memory_space=pltpu.VMEM),
        scratch_shapes=[
            pltpu.VMEM((T_PER, D), jnp.float32),
            pltpu.VMEM((T_PER, D), jnp.bfloat16),
            pltpu.VMEM((T_PER, D), jnp.bfloat16),
            pltpu.VMEM((T_PER, D), jnp.bfloat16),
            pltpu.VMEM((T_PER, D), jnp.bfloat16),
            pltpu.VMEM((T_PER, D), jnp.bfloat16),
            pltpu.VMEM((T_PER, D), jnp.bfloat16),
            pltpu.SemaphoreType.DMA((3 * N_PIPE,)),
            pltpu.SemaphoreType.DMA((3 * N_PIPE,)),
        ],
        compiler_params=pltpu.CompilerParams(collective_id=1),
    )(yflat, s2)


def kernel(x, router, W1, W2):
    my_y = lax.axis_index("y")

    xfull, idxfull, wfull = _gate_exch(x, router.T)
    xf = xfull.reshape(2 * T_PER, D)
    idx_g = jnp.concatenate([idxfull[0].T, idxfull[1].T], axis=0)
    w_g = jnp.concatenate([wfull[0].T, wfull[1].T], axis=0)

    base = my_y * E_LOCAL
    ee = idx_g.reshape(-1)
    ww = w_g.reshape(-1)
    le = ee - base
    is_local = (le >= 0) & (le < E_LOCAL)
    lec = jnp.clip(le, 0, E_LOCAL - 1)
    onehot = (lec[:, None] == jnp.arange(E_LOCAL)[None, :]) & is_local[:, None]
    pos = jnp.cumsum(onehot.astype(jnp.int32), axis=0) - 1
    pos_a = jnp.sum(jnp.where(onehot, pos, 0), axis=1)
    valid = is_local & (pos_a < C)
    slot = jnp.where(valid, lec * C + pos_a, N_SLOTS)
    s2 = slot.reshape(2 * T_PER, K).T
    w2 = w_g.T.astype(jnp.float32)

    fp = (lax.axis_index("x") * 2 + lax.axis_index("z")).astype(jnp.int32)
    yg = _ffn(fp[None], xf, s2, w2, W1, W2)

    return _combine(yg.reshape(N_SLOTS, D), s2)


# device time: 140852 ns/iter; 2.1127x vs baseline; 1.0054x over previous
import jax
import jax.numpy as jnp
from jax import lax
from jax.experimental import pallas as pl
from jax.experimental.pallas import tpu as pltpu

T_PER = 1024
D = 1024
F = 4096
E_LOCAL = 8
K = 2
C = 320
N_SLOTS = E_LOCAL * C
FB = 512
NF = F // FB


def _peer():
    return (lax.axis_index("x"), 1 - lax.axis_index("y"), lax.axis_index("z"))


def _neighbor_barrier(peer):
    barrier_sem = pltpu.get_barrier_semaphore()
    pl.semaphore_signal(
        barrier_sem, inc=1, device_id=peer, device_id_type=pl.DeviceIdType.MESH
    )
    pl.semaphore_wait(barrier_sem, 1)


N_E = 2 * E_LOCAL


def _gate_exch_body(
    x_ref, rt_ref, xfull_ref, idxfull_ref, wfull_ref, rfull_ref,
    send_sems, recv_sems
):
    my_y = lax.axis_index("y")
    peer = _peer()
    _neighbor_barrier(peer)

    def exch(i, ref):
        return pltpu.make_async_remote_copy(
            src_ref=ref.at[my_y],
            dst_ref=ref.at[my_y],
            send_sem=send_sems.at[i],
            recv_sem=recv_sems.at[i],
            device_id=peer,
            device_id_type=pl.DeviceIdType.MESH,
        )

    xfull_ref[pl.ds(my_y, 1)] = x_ref[...].astype(jnp.bfloat16)[None]
    rd_x = exch(0, xfull_ref)
    rd_x.start()

    rfull_ref[pl.ds(my_y, 1)] = rt_ref[...][None]
    rd_r = exch(1, rfull_ref)
    rd_r.start()
    rd_r.wait()

    rcat = jnp.concatenate([rfull_ref[0], rfull_ref[1]], axis=0)
    g = lax.dot_general(
        x_ref[...], rcat, (((1,), (1,)), ((), ())),
        preferred_element_type=jnp.float32,
        precision=lax.Precision.HIGHEST,
    )
    iota16 = lax.broadcasted_iota(jnp.int32, (T_PER, N_E), 1)
    m1 = jnp.max(g, axis=1)
    i1 = jnp.argmax(g, axis=1).astype(jnp.int32)
    g2 = jnp.where(iota16 == i1[:, None], -jnp.inf, g)
    m2 = jnp.max(g2, axis=1)
    i2 = jnp.argmax(g2, axis=1).astype(jnp.int32)
    w1 = 1.0 / (1.0 + jnp.exp(m2 - m1))

    idxfull_ref[pl.ds(my_y, 1)] = jnp.concatenate(
        [i1[None, None, :], i2[None, None, :]], axis=1
    )
    wfull_ref[pl.ds(my_y, 1)] = jnp.concatenate(
        [w1[None, None, :], (1.0 - w1)[None, None, :]], axis=1
    )
    rd_i = exch(2, idxfull_ref)
    rd_w = exch(3, wfull_ref)
    rd_i.start()
    rd_w.start()
    rd_i.wait()
    rd_w.wait()
    rd_x.wait()


def _gate_exch(x, rt):
    return pl.pallas_call(
        _gate_exch_body,
        out_shape=(
            jax.ShapeDtypeStruct((2, T_PER, D), jnp.bfloat16),
            jax.ShapeDtypeStruct((2, K, T_PER), jnp.int32),
            jax.ShapeDtypeStruct((2, K, T_PER), jnp.float32),
        ),
        in_specs=[pl.BlockSpec(memory_space=pltpu.VMEM)] * 2,
        out_specs=(pl.BlockSpec(memory_space=pltpu.VMEM),) * 3,
        scratch_shapes=[
            pltpu.VMEM((2, E_LOCAL, D), jnp.float32),
            pltpu.SemaphoreType.DMA((4,)),
            pltpu.SemaphoreType.DMA((4,)),
        ],
        compiler_params=pltpu.CompilerParams(collective_id=0),
    )(x, rt)


A = 2 * T_PER * K

N_FP = 4
NF_PER = NF // N_FP


def _ffn_body(fp_ref, xf_ref, s2_ref, w2_ref_in, w1_ref, w2_ref, out_ref,
              xg_ref, acc_ref, wg_ref):
    e = pl.program_id(0)
    f = pl.program_id(1)

    @pl.when(f == 0)
    def _():
        iota_c = lax.broadcasted_iota(jnp.int32, (C, 2 * T_PER), 0) + e * C
        oh0 = iota_c == s2_ref[0][None, :]
        oh1 = iota_c == s2_ref[1][None, :]
        ohc = oh0.astype(jnp.bfloat16) + oh1.astype(jnp.bfloat16)
        xg_ref[...] = jnp.dot(
            ohc, xf_ref[...], preferred_element_type=jnp.float32
        ).astype(jnp.bfloat16)
        wg_ref[0, :] = jnp.sum(
            jnp.where(oh0, w2_ref_in[0][None, :], 0.0)
            + jnp.where(oh1, w2_ref_in[1][None, :], 0.0),
            axis=1,
        )
        acc_ref[...] = jnp.zeros_like(acc_ref)

    h = jnp.dot(
        xg_ref[...],
        w1_ref[0].astype(jnp.bfloat16),
        preferred_element_type=jnp.float32,
    )
    h = jnp.maximum(h, 0.0).astype(jnp.bfloat16)
    acc_ref[...] += jnp.dot(
        h, w2_ref[0].astype(jnp.bfloat16), preferred_element_type=jnp.float32
    )

    @pl.when(f == NF_PER - 1)
    def _():
        out_ref[0] = (acc_ref[...] * wg_ref[0, :][:, None]).astype(jnp.bfloat16)


def _ffn(fp, xf_bf, s2, w2, W1, W2):
    grid_spec = pltpu.PrefetchScalarGridSpec(
        num_scalar_prefetch=1,
        grid=(E_LOCAL, NF_PER),
        in_specs=[
            pl.BlockSpec((2 * T_PER, D), lambda e, f, fp: (0, 0)),
            pl.BlockSpec((K, 2 * T_PER), lambda e, f, fp: (0, 0)),
            pl.BlockSpec((K, 2 * T_PER), lambda e, f, fp: (0, 0)),
            pl.BlockSpec((1, D, FB), lambda e, f, fp: (e, 0, fp[0] * NF_PER + f)),
            pl.BlockSpec((1, FB, D), lambda e, f, fp: (e, fp[0] * NF_PER + f, 0)),
        ],
        out_specs=pl.BlockSpec((1, C, D), lambda e, f, fp: (e, 0, 0)),
        scratch_shapes=[
            pltpu.VMEM((C, D), jnp.bfloat16),
            pltpu.VMEM((C, D), jnp.float32),
            pltpu.VMEM((1, C), jnp.float32),
        ],
    )
    return pl.pallas_call(
        _ffn_body,
        grid_spec=grid_spec,
        out_shape=jax.ShapeDtypeStruct((E_LOCAL, C, D), jnp.bfloat16),
    )(fp, xf_bf, s2, w2, W1, W2)


N_PIPE = 8
SEG = T_PER // N_PIPE


def _undispatch_seg(s2_ref, y_ref, base):
    st0 = s2_ref[0, pl.ds(base, SEG)]
    st1 = s2_ref[1, pl.ds(base, SEG)]
    iota = lax.broadcasted_iota(jnp.int32, (SEG, N_SLOTS), 1)
    oh2 = (st0[:, None] == iota).astype(jnp.bfloat16) + (
        st1[:, None] == iota
    ).astype(jnp.bfloat16)
    return jnp.dot(oh2, y_ref[...], preferred_element_type=jnp.float32)


def _combine_body(
    y_ref, s2_ref, out_ref, acc_ref, send_y, send_x, send_z,
    rbuf_y, rbuf_x, rbuf_z, send_sems, recv_sems
):
    my_x = lax.axis_index("x")
    my_y = lax.axis_index("y")
    my_z = lax.axis_index("z")
    n_y = (my_x, 1 - my_y, my_z)
    n_x = (1 - my_x, my_y, my_z)
    n_z = (my_x, my_y, 1 - my_z)

    barrier_sem = pltpu.get_barrier_semaphore()
    for nbr in (n_y, n_x, n_z):
        pl.semaphore_signal(
            barrier_sem, inc=1, device_id=nbr,
            device_id_type=pl.DeviceIdType.MESH,
        )
    pl.semaphore_wait(barrier_sem, 3)

    def exchange(sem_i, src, dst, q, nbr):
        return pltpu.make_async_remote_copy(
            src_ref=src.at[pl.ds(q * SEG, SEG)],
            dst_ref=dst.at[pl.ds(q * SEG, SEG)],
            send_sem=send_sems.at[sem_i],
            recv_sem=recv_sems.at[sem_i],
            device_id=nbr,
            device_id_type=pl.DeviceIdType.MESH,
        )

    rd_y = [None] * N_PIPE
    rd_x = [None] * N_PIPE
    rd_z = [None] * N_PIPE

    for q in range(N_PIPE):
        send_y[pl.ds(q * SEG, SEG), :] = _undispatch_seg(
            s2_ref, y_ref, (1 - my_y) * T_PER + q * SEG
        ).astype(jnp.bfloat16)
        rd_y[q] = exchange(q, send_y, rbuf_y, q, n_y)
        rd_y[q].start()

    for q in range(N_PIPE):
        acc_ref[pl.ds(q * SEG, SEG), :] = _undispatch_seg(
            s2_ref, y_ref, my_y * T_PER + q * SEG
        )

    for q in range(N_PIPE):
        rd_y[q].wait()
        rows = pl.ds(q * SEG, SEG)
        acc_ref[rows, :] += rbuf_y[rows, :].astype(jnp.float32)
        send_x[rows, :] = acc_ref[rows, :].astype(jnp.bfloat16)
        rd_x[q] = exchange(N_PIPE + q, send_x, rbuf_x, q, n_x)
        rd_x[q].start()

    for q in range(N_PIPE):
        rd_x[q].wait()
        rows = pl.ds(q * SEG, SEG)
        acc_ref[rows, :] += rbuf_x[rows, :].astype(jnp.float32)
        send_z[rows, :] = acc_ref[rows, :].astype(jnp.bfloat16)
        rd_z[q] = exchange(2 * N_PIPE + q, send_z, rbuf_z, q, n_z)
        rd_z[q].start()

    for q in range(N_PIPE):
        rd_z[q].wait()
        rows = pl.ds(q * SEG, SEG)
        out_ref[rows, :] = acc_ref[rows, :] + rbuf_z[rows, :].astype(
            jnp.float32
        )


def _combine(yflat, s2):
    return pl.pallas_call(
        _combine_body,
        out_shape=jax.ShapeDtypeStruct((T_PER, D), jnp.float32),
        in_specs=[
            pl.BlockSpec(memory_space=pltpu.VMEM),
            pl.BlockSpec(memory_space=pltpu.VMEM),
        ],
        out_specs=pl.BlockSpec(memory_space=pltpu.VMEM),
        scratch_shapes=[
            pltpu.VMEM((T_PER, D), jnp.float32),
            pltpu.VMEM((T_PER, D), jnp.bfloat16),
            pltpu.VMEM((T_PER, D), jnp.bfloat16),
            pltpu.VMEM((T_PER, D), jnp.bfloat16),
            pltpu.VMEM((T_PER, D), jnp.bfloat16),
            pltpu.VMEM((T_PER, D), jnp.bfloat16),
            pltpu.VMEM((T_PER, D), jnp.bfloat16),
            pltpu.SemaphoreType.DMA((3 * N_PIPE,)),
            pltpu.SemaphoreType.DMA((3 * N_PIPE,)),
        ],
        compiler_params=pltpu.CompilerParams(collective_id=1),
    )(yflat, s2)


def kernel(x, router, W1, W2):
    my_y = lax.axis_index("y")

    xfull, idxfull, wfull = _gate_exch(x, router.T)
    xf = xfull.reshape(2 * T_PER, D)
    idx_g = jnp.concatenate([idxfull[0].T, idxfull[1].T], axis=0)
    w_g = jnp.concatenate([wfull[0].T, wfull[1].T], axis=0)

    base = my_y * E_LOCAL
    ee = idx_g.reshape(-1)
    ww = w_g.reshape(-1)
    le = ee - base
    is_local = (le >= 0) & (le < E_LOCAL)
    lec = jnp.clip(le, 0, E_LOCAL - 1)
    onehot = (lec[:, None] == jnp.arange(E_LOCAL)[None, :]) & is_local[:, None]
    pos = jnp.cumsum(onehot.astype(jnp.int32), axis=0) - 1
    pos_a = jnp.sum(jnp.where(onehot, pos, 0), axis=1)
    valid = is_local & (pos_a < C)
    slot = jnp.where(valid, lec * C + pos_a, N_SLOTS)
    s2 = slot.reshape(2 * T_PER, K).T
    w2 = w_g.T.astype(jnp.float32)

    fp = (lax.axis_index("x") * 2 + lax.axis_index("z")).astype(jnp.int32)
    yg = _ffn(fp[None], xf, s2, w2, W1, W2)

    return _combine(yg.reshape(N_SLOTS, D), s2)
